# Initial kernel scaffold; baseline (speedup 1.0000x reference)
import jax
import jax.numpy as jnp
from jax.experimental import pallas as pl


def kernel(x, edge_index, W1, a_src1, a_dst1, b1, W2, a_src2, a_dst2, b2):
    raise NotImplementedError("write your pallas kernel here")


# speedup vs baseline: 34.6414x; 34.6414x over previous
import functools

import jax
import jax.numpy as jnp
from jax import lax
from jax.experimental import pallas as pl
from jax.experimental.pallas import tpu as pltpu
from jax.experimental.pallas import tpu_sc as plsc

N_NODES = 10000
NPAD = 10240
E_RAW = 320000
EP = 330240
EPW = EP // 32
KC = 120
NCHUNKS = EPW // KC
ROWS_PER_TILE = NPAD // 16

F1 = 128
F2 = 16


def _lrelu(v):
    return jnp.maximum(v, 0.2 * v)


def _splat(vec, i):
    return jnp.full((16,), vec[i], vec.dtype)


def _tc1_body(x_ref, w1_ref, ms_ref, md_ref, h_ref, as_ref, ad_ref,
              maxs_ref, maxd_ref):
    i = pl.program_id(0)
    h = jnp.dot(x_ref[...], w1_ref[...], preferred_element_type=jnp.float32)
    h_ref[...] = h
    a_s = jnp.dot(h, ms_ref[...], preferred_element_type=jnp.float32)
    a_d = jnp.dot(h, md_ref[...], preferred_element_type=jnp.float32)
    as_ref[...] = a_s
    ad_ref[...] = a_d

    @pl.when(i == 0)
    def _():
        maxs_ref[...] = jnp.full((8, 16), -3e38, jnp.float32)
        maxd_ref[...] = jnp.full((8, 16), -3e38, jnp.float32)

    bs = jnp.broadcast_to(jnp.max(a_s, axis=0, keepdims=True), (8, 16))
    bd = jnp.broadcast_to(jnp.max(a_d, axis=0, keepdims=True), (8, 16))
    maxs_ref[...] = jnp.maximum(maxs_ref[...], bs)
    maxd_ref[...] = jnp.maximum(maxd_ref[...], bd)


def _tc1(x, w1, ms, md):
    bn = 256
    grid = NPAD // bn
    return pl.pallas_call(
        _tc1_body,
        grid=(grid,),
        in_specs=[
            pl.BlockSpec((bn, 128), lambda i: (i, 0)),
            pl.BlockSpec((128, 128), lambda i: (0, 0)),
            pl.BlockSpec((128, 16), lambda i: (0, 0)),
            pl.BlockSpec((128, 16), lambda i: (0, 0)),
        ],
        out_specs=[
            pl.BlockSpec((bn, 128), lambda i: (i, 0)),
            pl.BlockSpec((bn, 16), lambda i: (i, 0)),
            pl.BlockSpec((bn, 16), lambda i: (i, 0)),
            pl.BlockSpec((8, 16), lambda i: (0, 0)),
            pl.BlockSpec((8, 16), lambda i: (0, 0)),
        ],
        out_shape=[
            jax.ShapeDtypeStruct((NPAD, 128), jnp.float32),
            jax.ShapeDtypeStruct((NPAD, 16), jnp.float32),
            jax.ShapeDtypeStruct((NPAD, 16), jnp.float32),
            jax.ShapeDtypeStruct((8, 16), jnp.float32),
            jax.ShapeDtypeStruct((8, 16), jnp.float32),
        ],
    )(x, w1, ms, md)


def _inv_body(d_ref, o_ref):
    o_ref[...] = 1.0 / (d_ref[0] + d_ref[1] + 1e-16)


def _tc_inv(denom2):
    bn = 512
    return pl.pallas_call(
        _inv_body,
        grid=(NPAD // bn,),
        in_specs=[pl.BlockSpec((2, bn, 16), lambda i: (0, i, 0))],
        out_specs=pl.BlockSpec((bn, 16), lambda i: (i, 0)),
        out_shape=jax.ShapeDtypeStruct((NPAD, 16), jnp.float32),
    )(denom2)


def _tc3_body(o_ref, b1_ref, w2p_ref, ms2_ref, md2_ref,
              h2_ref, as2_ref, ad2_ref, maxs_ref, maxd_ref):
    i = pl.program_id(0)
    t = o_ref[0] + o_ref[1] + b1_ref[0:1, :]
    h2pre = jnp.where(t > 0, t, jnp.exp(jnp.minimum(t, 0.0)) - 1.0)
    h2 = jnp.dot(h2pre, w2p_ref[...], preferred_element_type=jnp.float32)
    a_s = jnp.dot(h2pre, ms2_ref[...], preferred_element_type=jnp.float32)
    a_d = jnp.dot(h2pre, md2_ref[...], preferred_element_type=jnp.float32)
    h2_ref[...] = h2
    as2_ref[...] = a_s
    ad2_ref[...] = a_d

    @pl.when(i == 0)
    def _():
        maxs_ref[...] = jnp.full((8, 16), -3e38, jnp.float32)
        maxd_ref[...] = jnp.full((8, 16), -3e38, jnp.float32)

    bs = jnp.broadcast_to(jnp.max(a_s, axis=0, keepdims=True), (8, 16))
    bd = jnp.broadcast_to(jnp.max(a_d, axis=0, keepdims=True), (8, 16))
    maxs_ref[...] = jnp.maximum(maxs_ref[...], bs)
    maxd_ref[...] = jnp.maximum(maxd_ref[...], bd)


def _tc3(out2c, b1b, w2p, ms2, md2):
    bn = 256
    return pl.pallas_call(
        _tc3_body,
        grid=(NPAD // bn,),
        in_specs=[
            pl.BlockSpec((2, bn, 128), lambda i: (0, i, 0)),
            pl.BlockSpec((8, 128), lambda i: (0, 0)),
            pl.BlockSpec((128, 16), lambda i: (0, 0)),
            pl.BlockSpec((128, 16), lambda i: (0, 0)),
            pl.BlockSpec((128, 16), lambda i: (0, 0)),
        ],
        out_specs=[
            pl.BlockSpec((bn, 16), lambda i: (i, 0)),
            pl.BlockSpec((bn, 16), lambda i: (i, 0)),
            pl.BlockSpec((bn, 16), lambda i: (i, 0)),
            pl.BlockSpec((8, 16), lambda i: (0, 0)),
            pl.BlockSpec((8, 16), lambda i: (0, 0)),
        ],
        out_shape=[
            jax.ShapeDtypeStruct((NPAD, 16), jnp.float32),
            jax.ShapeDtypeStruct((NPAD, 16), jnp.float32),
            jax.ShapeDtypeStruct((NPAD, 16), jnp.float32),
            jax.ShapeDtypeStruct((8, 16), jnp.float32),
            jax.ShapeDtypeStruct((8, 16), jnp.float32),
        ],
    )(out2c, b1b, w2p, ms2, md2)


def _tc4_body(o_ref, b2_ref, out_ref):
    t = o_ref[0] + o_ref[1] + b2_ref[0:1, :]
    lane = lax.broadcasted_iota(jnp.int32, t.shape, 1)
    valid = lane < 7
    tm = jnp.where(valid, t, -3e38)
    m = jnp.max(tm, axis=1, keepdims=True)
    s = jnp.sum(jnp.where(valid, jnp.exp(t - m), 0.0), axis=1, keepdims=True)
    out_ref[...] = t - m - jnp.log(s)


def _tc4(out2c, b2b):
    bn = 512
    return pl.pallas_call(
        _tc4_body,
        grid=(NPAD // bn,),
        in_specs=[
            pl.BlockSpec((2, bn, 16), lambda i: (0, i, 0)),
            pl.BlockSpec((8, 16), lambda i: (0, 0)),
        ],
        out_specs=pl.BlockSpec((bn, 16), lambda i: (i, 0)),
        out_shape=jax.ShapeDtypeStruct((NPAD, 16), jnp.float32),
    )(out2c, b2b)


_MESH = plsc.VectorSubcoreMesh(core_axis_name="c", subcore_axis_name="s")
_SC_PARAMS = pltpu.CompilerParams(use_tc_tiling_on_sc=False)


@functools.partial(
    pl.kernel,
    mesh=_MESH,
    compiler_params=_SC_PARAMS,
    out_type=[
        jax.ShapeDtypeStruct((EP, 16), jnp.float32),
        jax.ShapeDtypeStruct((2, NPAD, 16), jnp.float32),
    ],
    scratch_types=[
        pltpu.VMEM((KC,), jnp.int32),
        pltpu.VMEM((KC,), jnp.int32),
        pltpu.VMEM((KC, 16), jnp.float32),
        pltpu.VMEM((KC, 16), jnp.float32),
        pltpu.VMEM((KC, 16), jnp.float32),
        pltpu.VMEM((16,), jnp.float32),
        pltpu.VMEM((16,), jnp.float32),
        pltpu.VMEM_SHARED((NPAD, 16), jnp.float32),
        pltpu.SemaphoreType.DMA,
    ],
)
def _sc_pass_a(src_hbm, dst_hbm, asrc_hbm, adst_hbm, maxs_hbm, maxd_hbm,
               zeros_hbm, ex_hbm, denom_hbm,
               src_v, dst_v, arows, drows, exbuf, m1, m2, denom_sp, sem):
    cid = lax.axis_index("c")
    sid = lax.axis_index("s")
    wid = cid * 16 + sid

    pltpu.sync_copy(zeros_hbm.at[pl.ds(sid * ROWS_PER_TILE, ROWS_PER_TILE)],
                    denom_sp.at[pl.ds(sid * ROWS_PER_TILE, ROWS_PER_TILE)])
    pltpu.sync_copy(maxs_hbm.at[0], m1)
    pltpu.sync_copy(maxd_hbm.at[0], m2)
    sub = _lrelu(m1[...] + m2[...])
    plsc.subcore_barrier()

    def chunk(c, carry):
        base = pl.multiple_of(wid * EPW + c * KC, 8)
        pltpu.sync_copy(src_hbm.at[pl.ds(base, KC)], src_v)
        pltpu.sync_copy(dst_hbm.at[pl.ds(base, KC)], dst_v)
        pltpu.async_copy(asrc_hbm.at[src_v], arows, sem).wait()
        pltpu.async_copy(adst_hbm.at[dst_v], drows, sem).wait()

        def edge(e, carry2):
            v = _lrelu(arows[e, :] + drows[e, :])
            exbuf[e, :] = jnp.exp(v - sub)
            return carry2

        lax.fori_loop(0, KC, edge, 0)
        pltpu.sync_copy(exbuf, denom_sp.at[dst_v], add=True)
        pltpu.sync_copy(exbuf, ex_hbm.at[pl.ds(base, KC)])
        return carry

    lax.fori_loop(0, NCHUNKS, chunk, 0)
    plsc.subcore_barrier()
    pltpu.sync_copy(denom_sp.at[pl.ds(sid * ROWS_PER_TILE, ROWS_PER_TILE)],
                    denom_hbm.at[cid, pl.ds(sid * ROWS_PER_TILE, ROWS_PER_TILE)])


def _make_sc_pass_b(width):
    nheads = width // 16

    @functools.partial(
        pl.kernel,
        mesh=_MESH,
        compiler_params=_SC_PARAMS,
        out_type=jax.ShapeDtypeStruct((2, NPAD, width), jnp.float32),
        scratch_types=[
            pltpu.VMEM((KC,), jnp.int32),
            pltpu.VMEM((KC,), jnp.int32),
            pltpu.VMEM((KC, width), jnp.float32),
            pltpu.VMEM((KC, 16), jnp.float32),
            pltpu.VMEM((KC, 16), jnp.float32),
            pltpu.VMEM_SHARED((NPAD, width), jnp.float32),
            pltpu.SemaphoreType.DMA,
        ],
    )
    def _sc_pass_b(src_hbm, dst_hbm, ex_hbm, inv_hbm, h_hbm, zeros_hbm,
                   out_hbm, src_v, dst_v, hrows, exrows, invrows, out_sp, sem):
        cid = lax.axis_index("c")
        sid = lax.axis_index("s")
        wid = cid * 16 + sid

        pltpu.sync_copy(zeros_hbm.at[pl.ds(sid * ROWS_PER_TILE, ROWS_PER_TILE)],
                        out_sp.at[pl.ds(sid * ROWS_PER_TILE, ROWS_PER_TILE)])
        plsc.subcore_barrier()

        def chunk(c, carry):
            base = pl.multiple_of(wid * EPW + c * KC, 8)
            pltpu.sync_copy(src_hbm.at[pl.ds(base, KC)], src_v)
            pltpu.sync_copy(dst_hbm.at[pl.ds(base, KC)], dst_v)
            pltpu.async_copy(h_hbm.at[src_v], hrows, sem).wait()
            pltpu.async_copy(inv_hbm.at[dst_v], invrows, sem).wait()
            pltpu.sync_copy(ex_hbm.at[pl.ds(base, KC)], exrows)

            def edge(e, carry2):
                coef = exrows[e, :] * invrows[e, :]
                for hd in range(nheads):
                    hv = hrows[e, pl.ds(hd * 16, 16)]
                    hrows[e, pl.ds(hd * 16, 16)] = hv * _splat(coef, hd)
                return carry2

            lax.fori_loop(0, KC, edge, 0)
            pltpu.sync_copy(hrows, out_sp.at[dst_v], add=True)
            return carry

        lax.fori_loop(0, NCHUNKS, chunk, 0)
        plsc.subcore_barrier()
        pltpu.sync_copy(out_sp.at[pl.ds(sid * ROWS_PER_TILE, ROWS_PER_TILE)],
                        out_hbm.at[cid, pl.ds(sid * ROWS_PER_TILE, ROWS_PER_TILE)])

    return _sc_pass_b


_sc_pass_b128 = _make_sc_pass_b(128)
_sc_pass_b16 = _make_sc_pass_b(16)


def kernel(x, edge_index, W1, a_src1, a_dst1, b1, W2, a_src2, a_dst2, b2):
    n = N_NODES
    loop = jnp.arange(n, dtype=jnp.int32)
    pad_e = EP - (E_RAW + n)
    padv = jnp.full((pad_e,), n, dtype=jnp.int32)
    srcp = jnp.concatenate([edge_index[0], loop, padv])
    dstp = jnp.concatenate([edge_index[1], loop, padv])

    xp = jnp.zeros((NPAD, 128), jnp.float32).at[:n].set(x)

    hd_ids = jnp.repeat(jnp.arange(8), 16)
    sel = (hd_ids[:, None] == jnp.arange(8)[None, :])
    ms1 = jnp.pad(jnp.where(sel, a_src1.reshape(-1)[:, None], 0.0), ((0, 0), (0, 8)))
    md1 = jnp.pad(jnp.where(sel, a_dst1.reshape(-1)[:, None], 0.0), ((0, 0), (0, 8)))

    w2p = jnp.pad(W2, ((0, 0), (0, 16 - W2.shape[1])))
    ms2 = jnp.pad(W2 @ a_src2.T, ((0, 0), (0, 15)))
    md2 = jnp.pad(W2 @ a_dst2.T, ((0, 0), (0, 15)))
    b1b = jnp.broadcast_to(b1[None, :], (8, 128))
    b2b = jnp.broadcast_to(jnp.pad(b2, (0, 16 - b2.shape[0]))[None, :], (8, 16))
    zeros128 = jnp.zeros((NPAD, 128), jnp.float32)
    zeros16 = jnp.zeros((NPAD, 16), jnp.float32)

    h1, as1, ad1, maxs1, maxd1 = _tc1(xp, W1, ms1, md1)
    ex1, denom1 = _sc_pass_a(srcp, dstp, as1, ad1, maxs1, maxd1, zeros16)
    inv1 = _tc_inv(denom1)
    out1 = _sc_pass_b128(srcp, dstp, ex1, inv1, h1, zeros128)

    h2, as2, ad2, maxs2, maxd2 = _tc3(out1, b1b, w2p, ms2, md2)
    ex2, denom2 = _sc_pass_a(srcp, dstp, as2, ad2, maxs2, maxd2, zeros16)
    inv2 = _tc_inv(denom2)
    out2 = _sc_pass_b16(srcp, dstp, ex2, inv2, h2, zeros16)

    res = _tc4(out2, b2b)
    return res[:n, :7]


# speedup vs baseline: 37.6140x; 1.0858x over previous
import functools

import jax
import jax.numpy as jnp
from jax import lax
from jax.experimental import pallas as pl
from jax.experimental.pallas import tpu as pltpu
from jax.experimental.pallas import tpu_sc as plsc

N_NODES = 10000
NPAD = 10240
E_RAW = 320000
EP = 330240
EPW = EP // 32
KC = 120
NCHUNKS = EPW // KC
ROWS_PER_TILE = NPAD // 16

F1 = 128
F2 = 16


def _lrelu(v):
    return jnp.maximum(v, 0.2 * v)


def _splat(vec, i):
    return jnp.full((16,), vec[i], vec.dtype)


def _tc1_body(x_ref, w1_ref, ms_ref, md_ref, h_ref, as_ref, ad_ref,
              maxs_ref, maxd_ref):
    i = pl.program_id(0)
    h = jnp.dot(x_ref[...], w1_ref[...], preferred_element_type=jnp.float32)
    h_ref[...] = h
    a_s = jnp.dot(h, ms_ref[...], preferred_element_type=jnp.float32)
    a_d = jnp.dot(h, md_ref[...], preferred_element_type=jnp.float32)
    as_ref[...] = a_s
    ad_ref[...] = a_d

    @pl.when(i == 0)
    def _():
        maxs_ref[...] = jnp.full((8, 16), -3e38, jnp.float32)
        maxd_ref[...] = jnp.full((8, 16), -3e38, jnp.float32)

    bs = jnp.broadcast_to(jnp.max(a_s, axis=0, keepdims=True), (8, 16))
    bd = jnp.broadcast_to(jnp.max(a_d, axis=0, keepdims=True), (8, 16))
    maxs_ref[...] = jnp.maximum(maxs_ref[...], bs)
    maxd_ref[...] = jnp.maximum(maxd_ref[...], bd)


def _tc1(x, w1, ms, md):
    bn = 256
    grid = NPAD // bn
    return pl.pallas_call(
        _tc1_body,
        grid=(grid,),
        in_specs=[
            pl.BlockSpec((bn, 128), lambda i: (i, 0)),
            pl.BlockSpec((128, 128), lambda i: (0, 0)),
            pl.BlockSpec((128, 16), lambda i: (0, 0)),
            pl.BlockSpec((128, 16), lambda i: (0, 0)),
        ],
        out_specs=[
            pl.BlockSpec((bn, 128), lambda i: (i, 0)),
            pl.BlockSpec((bn, 16), lambda i: (i, 0)),
            pl.BlockSpec((bn, 16), lambda i: (i, 0)),
            pl.BlockSpec((8, 16), lambda i: (0, 0)),
            pl.BlockSpec((8, 16), lambda i: (0, 0)),
        ],
        out_shape=[
            jax.ShapeDtypeStruct((NPAD, 128), jnp.float32),
            jax.ShapeDtypeStruct((NPAD, 16), jnp.float32),
            jax.ShapeDtypeStruct((NPAD, 16), jnp.float32),
            jax.ShapeDtypeStruct((8, 16), jnp.float32),
            jax.ShapeDtypeStruct((8, 16), jnp.float32),
        ],
    )(x, w1, ms, md)


def _inv_body(d_ref, o_ref):
    o_ref[...] = 1.0 / (d_ref[0] + d_ref[1] + 1e-16)


def _tc_inv(denom2):
    bn = 512
    return pl.pallas_call(
        _inv_body,
        grid=(NPAD // bn,),
        in_specs=[pl.BlockSpec((2, bn, 16), lambda i: (0, i, 0))],
        out_specs=pl.BlockSpec((bn, 16), lambda i: (i, 0)),
        out_shape=jax.ShapeDtypeStruct((NPAD, 16), jnp.float32),
    )(denom2)


def _tc3_body(o_ref, b1_ref, w2p_ref, ms2_ref, md2_ref,
              h2_ref, as2_ref, ad2_ref, maxs_ref, maxd_ref):
    i = pl.program_id(0)
    t = o_ref[0] + o_ref[1] + b1_ref[0:1, :]
    h2pre = jnp.where(t > 0, t, jnp.exp(jnp.minimum(t, 0.0)) - 1.0)
    h2 = jnp.dot(h2pre, w2p_ref[...], preferred_element_type=jnp.float32)
    a_s = jnp.dot(h2pre, ms2_ref[...], preferred_element_type=jnp.float32)
    a_d = jnp.dot(h2pre, md2_ref[...], preferred_element_type=jnp.float32)
    h2_ref[...] = h2
    as2_ref[...] = a_s
    ad2_ref[...] = a_d

    @pl.when(i == 0)
    def _():
        maxs_ref[...] = jnp.full((8, 16), -3e38, jnp.float32)
        maxd_ref[...] = jnp.full((8, 16), -3e38, jnp.float32)

    bs = jnp.broadcast_to(jnp.max(a_s, axis=0, keepdims=True), (8, 16))
    bd = jnp.broadcast_to(jnp.max(a_d, axis=0, keepdims=True), (8, 16))
    maxs_ref[...] = jnp.maximum(maxs_ref[...], bs)
    maxd_ref[...] = jnp.maximum(maxd_ref[...], bd)


def _tc3(out2c, b1b, w2p, ms2, md2):
    bn = 256
    return pl.pallas_call(
        _tc3_body,
        grid=(NPAD // bn,),
        in_specs=[
            pl.BlockSpec((2, bn, 128), lambda i: (0, i, 0)),
            pl.BlockSpec((8, 128), lambda i: (0, 0)),
            pl.BlockSpec((128, 16), lambda i: (0, 0)),
            pl.BlockSpec((128, 16), lambda i: (0, 0)),
            pl.BlockSpec((128, 16), lambda i: (0, 0)),
        ],
        out_specs=[
            pl.BlockSpec((bn, 16), lambda i: (i, 0)),
            pl.BlockSpec((bn, 16), lambda i: (i, 0)),
            pl.BlockSpec((bn, 16), lambda i: (i, 0)),
            pl.BlockSpec((8, 16), lambda i: (0, 0)),
            pl.BlockSpec((8, 16), lambda i: (0, 0)),
        ],
        out_shape=[
            jax.ShapeDtypeStruct((NPAD, 16), jnp.float32),
            jax.ShapeDtypeStruct((NPAD, 16), jnp.float32),
            jax.ShapeDtypeStruct((NPAD, 16), jnp.float32),
            jax.ShapeDtypeStruct((8, 16), jnp.float32),
            jax.ShapeDtypeStruct((8, 16), jnp.float32),
        ],
    )(out2c, b1b, w2p, ms2, md2)


def _tc4_body(o_ref, b2_ref, out_ref):
    t = o_ref[0] + o_ref[1] + b2_ref[0:1, :]
    lane = lax.broadcasted_iota(jnp.int32, t.shape, 1)
    valid = lane < 7
    tm = jnp.where(valid, t, -3e38)
    m = jnp.max(tm, axis=1, keepdims=True)
    s = jnp.sum(jnp.where(valid, jnp.exp(t - m), 0.0), axis=1, keepdims=True)
    out_ref[...] = t - m - jnp.log(s)


def _tc4(out2c, b2b):
    bn = 512
    return pl.pallas_call(
        _tc4_body,
        grid=(NPAD // bn,),
        in_specs=[
            pl.BlockSpec((2, bn, 16), lambda i: (0, i, 0)),
            pl.BlockSpec((8, 16), lambda i: (0, 0)),
        ],
        out_specs=pl.BlockSpec((bn, 16), lambda i: (i, 0)),
        out_shape=jax.ShapeDtypeStruct((NPAD, 16), jnp.float32),
    )(out2c, b2b)


_MESH = plsc.VectorSubcoreMesh(core_axis_name="c", subcore_axis_name="s")
_SC_PARAMS = pltpu.CompilerParams(use_tc_tiling_on_sc=False)


@functools.partial(
    pl.kernel,
    mesh=_MESH,
    compiler_params=_SC_PARAMS,
    out_type=[
        jax.ShapeDtypeStruct((EP, 16), jnp.float32),
        jax.ShapeDtypeStruct((2, NPAD, 16), jnp.float32),
    ],
    scratch_types=[
        pltpu.VMEM((KC,), jnp.int32),
        pltpu.VMEM((KC,), jnp.int32),
        pltpu.VMEM((KC, 16), jnp.float32),
        pltpu.VMEM((KC, 16), jnp.float32),
        pltpu.VMEM((KC, 16), jnp.float32),
        pltpu.VMEM((16,), jnp.float32),
        pltpu.VMEM((16,), jnp.float32),
        pltpu.VMEM_SHARED((NPAD, 16), jnp.float32),
        pltpu.SemaphoreType.DMA,
    ],
)
def _sc_pass_a(src_hbm, dst_hbm, asrc_hbm, adst_hbm, maxs_hbm, maxd_hbm,
               zeros_hbm, ex_hbm, denom_hbm,
               src_v, dst_v, arows, drows, exbuf, m1, m2, denom_sp, sem):
    cid = lax.axis_index("c")
    sid = lax.axis_index("s")
    wid = cid * 16 + sid

    pltpu.sync_copy(zeros_hbm.at[pl.ds(sid * ROWS_PER_TILE, ROWS_PER_TILE)],
                    denom_sp.at[pl.ds(sid * ROWS_PER_TILE, ROWS_PER_TILE)])
    pltpu.sync_copy(maxs_hbm.at[0], m1)
    pltpu.sync_copy(maxd_hbm.at[0], m2)
    sub = _lrelu(m1[...] + m2[...])
    plsc.subcore_barrier()

    def chunk(c, carry):
        base = pl.multiple_of(wid * EPW + c * KC, 8)
        pltpu.sync_copy(src_hbm.at[pl.ds(base, KC)], src_v)
        pltpu.sync_copy(dst_hbm.at[pl.ds(base, KC)], dst_v)
        pltpu.async_copy(asrc_hbm.at[src_v], arows, sem).wait()
        pltpu.async_copy(adst_hbm.at[dst_v], drows, sem).wait()

        for e in range(KC):
            v = _lrelu(arows[e, :] + drows[e, :])
            exbuf[e, :] = jnp.exp(v - sub)
        pltpu.sync_copy(exbuf, denom_sp.at[dst_v], add=True)
        pltpu.sync_copy(exbuf, ex_hbm.at[pl.ds(base, KC)])
        return carry

    lax.fori_loop(0, NCHUNKS, chunk, 0)
    plsc.subcore_barrier()
    pltpu.sync_copy(denom_sp.at[pl.ds(sid * ROWS_PER_TILE, ROWS_PER_TILE)],
                    denom_hbm.at[cid, pl.ds(sid * ROWS_PER_TILE, ROWS_PER_TILE)])


def _make_sc_pass_b(width):
    nheads = width // 16

    @functools.partial(
        pl.kernel,
        mesh=_MESH,
        compiler_params=_SC_PARAMS,
        out_type=jax.ShapeDtypeStruct((2, NPAD, width), jnp.float32),
        scratch_types=[
            pltpu.VMEM((KC,), jnp.int32),
            pltpu.VMEM((KC,), jnp.int32),
            pltpu.VMEM((KC, width), jnp.float32),
            pltpu.VMEM((KC, 16), jnp.float32),
            pltpu.VMEM((KC, 16), jnp.float32),
            pltpu.VMEM_SHARED((NPAD, width), jnp.float32),
            pltpu.SemaphoreType.DMA,
        ],
    )
    def _sc_pass_b(src_hbm, dst_hbm, ex_hbm, inv_hbm, h_hbm, zeros_hbm,
                   out_hbm, src_v, dst_v, hrows, exrows, invrows, out_sp, sem):
        cid = lax.axis_index("c")
        sid = lax.axis_index("s")
        wid = cid * 16 + sid

        pltpu.sync_copy(zeros_hbm.at[pl.ds(sid * ROWS_PER_TILE, ROWS_PER_TILE)],
                        out_sp.at[pl.ds(sid * ROWS_PER_TILE, ROWS_PER_TILE)])
        plsc.subcore_barrier()

        def chunk(c, carry):
            base = pl.multiple_of(wid * EPW + c * KC, 8)
            pltpu.sync_copy(src_hbm.at[pl.ds(base, KC)], src_v)
            pltpu.sync_copy(dst_hbm.at[pl.ds(base, KC)], dst_v)
            pltpu.async_copy(h_hbm.at[src_v], hrows, sem).wait()
            pltpu.async_copy(inv_hbm.at[dst_v], invrows, sem).wait()
            pltpu.sync_copy(ex_hbm.at[pl.ds(base, KC)], exrows)

            for e in range(KC):
                coef = exrows[e, :] * invrows[e, :]
                for hd in range(nheads):
                    hv = hrows[e, pl.ds(hd * 16, 16)]
                    hrows[e, pl.ds(hd * 16, 16)] = hv * _splat(coef, hd)
            pltpu.sync_copy(hrows, out_sp.at[dst_v], add=True)
            return carry

        lax.fori_loop(0, NCHUNKS, chunk, 0)
        plsc.subcore_barrier()
        pltpu.sync_copy(out_sp.at[pl.ds(sid * ROWS_PER_TILE, ROWS_PER_TILE)],
                        out_hbm.at[cid, pl.ds(sid * ROWS_PER_TILE, ROWS_PER_TILE)])

    return _sc_pass_b


_sc_pass_b128 = _make_sc_pass_b(128)
_sc_pass_b16 = _make_sc_pass_b(16)


def kernel(x, edge_index, W1, a_src1, a_dst1, b1, W2, a_src2, a_dst2, b2):
    n = N_NODES
    loop = jnp.arange(n, dtype=jnp.int32)
    pad_e = EP - (E_RAW + n)
    padv = jnp.full((pad_e,), n, dtype=jnp.int32)
    srcp = jnp.concatenate([edge_index[0], loop, padv])
    dstp = jnp.concatenate([edge_index[1], loop, padv])

    xp = jnp.zeros((NPAD, 128), jnp.float32).at[:n].set(x)

    hd_ids = jnp.repeat(jnp.arange(8), 16)
    sel = (hd_ids[:, None] == jnp.arange(8)[None, :])
    ms1 = jnp.pad(jnp.where(sel, a_src1.reshape(-1)[:, None], 0.0), ((0, 0), (0, 8)))
    md1 = jnp.pad(jnp.where(sel, a_dst1.reshape(-1)[:, None], 0.0), ((0, 0), (0, 8)))

    w2p = jnp.pad(W2, ((0, 0), (0, 16 - W2.shape[1])))
    ms2 = jnp.pad(W2 @ a_src2.T, ((0, 0), (0, 15)))
    md2 = jnp.pad(W2 @ a_dst2.T, ((0, 0), (0, 15)))
    b1b = jnp.broadcast_to(b1[None, :], (8, 128))
    b2b = jnp.broadcast_to(jnp.pad(b2, (0, 16 - b2.shape[0]))[None, :], (8, 16))
    zeros128 = jnp.zeros((NPAD, 128), jnp.float32)
    zeros16 = jnp.zeros((NPAD, 16), jnp.float32)

    h1, as1, ad1, maxs1, maxd1 = _tc1(xp, W1, ms1, md1)
    ex1, denom1 = _sc_pass_a(srcp, dstp, as1, ad1, maxs1, maxd1, zeros16)
    inv1 = _tc_inv(denom1)
    out1 = _sc_pass_b128(srcp, dstp, ex1, inv1, h1, zeros128)

    h2, as2, ad2, maxs2, maxd2 = _tc3(out1, b1b, w2p, ms2, md2)
    ex2, denom2 = _sc_pass_a(srcp, dstp, as2, ad2, maxs2, maxd2, zeros16)
    inv2 = _tc_inv(denom2)
    out2 = _sc_pass_b16(srcp, dstp, ex2, inv2, h2, zeros16)

    res = _tc4(out2, b2b)
    return res[:n, :7]


# speedup vs baseline: 68.9642x; 1.8335x over previous
import functools

import jax
import jax.numpy as jnp
from jax import lax
from jax.experimental import pallas as pl
from jax.experimental.pallas import tpu as pltpu
from jax.experimental.pallas import tpu_sc as plsc

N_NODES = 10000
NPAD = 10240
E_RAW = 320000
EP = 331776
EPW = EP // 32
KC = 96
NCHUNKS = EPW // KC
NB = 4
SUPERS = NCHUNKS // NB
ROWS_PER_TILE = NPAD // 16

F1 = 128
F2 = 16


def _lrelu(v):
    return jnp.maximum(v, 0.2 * v)


def _splat(vec, i):
    return jnp.full((16,), vec[i], vec.dtype)


def _tc1_body(x_ref, w1_ref, ms_ref, md_ref, h_ref, as_ref, ad_ref,
              maxs_ref, maxd_ref):
    i = pl.program_id(0)
    h = jnp.dot(x_ref[...], w1_ref[...], preferred_element_type=jnp.float32)
    h_ref[...] = h
    a_s = jnp.dot(h, ms_ref[...], preferred_element_type=jnp.float32)
    a_d = jnp.dot(h, md_ref[...], preferred_element_type=jnp.float32)
    as_ref[...] = a_s
    ad_ref[...] = a_d

    @pl.when(i == 0)
    def _():
        maxs_ref[...] = jnp.full((8, 16), -3e38, jnp.float32)
        maxd_ref[...] = jnp.full((8, 16), -3e38, jnp.float32)

    bs = jnp.broadcast_to(jnp.max(a_s, axis=0, keepdims=True), (8, 16))
    bd = jnp.broadcast_to(jnp.max(a_d, axis=0, keepdims=True), (8, 16))
    maxs_ref[...] = jnp.maximum(maxs_ref[...], bs)
    maxd_ref[...] = jnp.maximum(maxd_ref[...], bd)


def _tc1(x, w1, ms, md):
    bn = 256
    grid = NPAD // bn
    return pl.pallas_call(
        _tc1_body,
        grid=(grid,),
        in_specs=[
            pl.BlockSpec((bn, 128), lambda i: (i, 0)),
            pl.BlockSpec((128, 128), lambda i: (0, 0)),
            pl.BlockSpec((128, 16), lambda i: (0, 0)),
            pl.BlockSpec((128, 16), lambda i: (0, 0)),
        ],
        out_specs=[
            pl.BlockSpec((bn, 128), lambda i: (i, 0)),
            pl.BlockSpec((bn, 16), lambda i: (i, 0)),
            pl.BlockSpec((bn, 16), lambda i: (i, 0)),
            pl.BlockSpec((8, 16), lambda i: (0, 0)),
            pl.BlockSpec((8, 16), lambda i: (0, 0)),
        ],
        out_shape=[
            jax.ShapeDtypeStruct((NPAD, 128), jnp.float32),
            jax.ShapeDtypeStruct((NPAD, 16), jnp.float32),
            jax.ShapeDtypeStruct((NPAD, 16), jnp.float32),
            jax.ShapeDtypeStruct((8, 16), jnp.float32),
            jax.ShapeDtypeStruct((8, 16), jnp.float32),
        ],
    )(x, w1, ms, md)


def _inv_body(d_ref, o_ref):
    o_ref[...] = 1.0 / (d_ref[0] + d_ref[1] + 1e-16)


def _tc_inv(denom2):
    bn = 512
    return pl.pallas_call(
        _inv_body,
        grid=(NPAD // bn,),
        in_specs=[pl.BlockSpec((2, bn, 16), lambda i: (0, i, 0))],
        out_specs=pl.BlockSpec((bn, 16), lambda i: (i, 0)),
        out_shape=jax.ShapeDtypeStruct((NPAD, 16), jnp.float32),
    )(denom2)


def _tc3_body(o_ref, b1_ref, w2p_ref, ms2_ref, md2_ref,
              h2_ref, as2_ref, ad2_ref, maxs_ref, maxd_ref):
    i = pl.program_id(0)
    t = o_ref[0] + o_ref[1] + b1_ref[0:1, :]
    h2pre = jnp.where(t > 0, t, jnp.exp(jnp.minimum(t, 0.0)) - 1.0)
    h2 = jnp.dot(h2pre, w2p_ref[...], preferred_element_type=jnp.float32)
    a_s = jnp.dot(h2pre, ms2_ref[...], preferred_element_type=jnp.float32)
    a_d = jnp.dot(h2pre, md2_ref[...], preferred_element_type=jnp.float32)
    h2_ref[...] = h2
    as2_ref[...] = a_s
    ad2_ref[...] = a_d

    @pl.when(i == 0)
    def _():
        maxs_ref[...] = jnp.full((8, 16), -3e38, jnp.float32)
        maxd_ref[...] = jnp.full((8, 16), -3e38, jnp.float32)

    bs = jnp.broadcast_to(jnp.max(a_s, axis=0, keepdims=True), (8, 16))
    bd = jnp.broadcast_to(jnp.max(a_d, axis=0, keepdims=True), (8, 16))
    maxs_ref[...] = jnp.maximum(maxs_ref[...], bs)
    maxd_ref[...] = jnp.maximum(maxd_ref[...], bd)


def _tc3(out2c, b1b, w2p, ms2, md2):
    bn = 256
    return pl.pallas_call(
        _tc3_body,
        grid=(NPAD // bn,),
        in_specs=[
            pl.BlockSpec((2, bn, 128), lambda i: (0, i, 0)),
            pl.BlockSpec((8, 128), lambda i: (0, 0)),
            pl.BlockSpec((128, 16), lambda i: (0, 0)),
            pl.BlockSpec((128, 16), lambda i: (0, 0)),
            pl.BlockSpec((128, 16), lambda i: (0, 0)),
        ],
        out_specs=[
            pl.BlockSpec((bn, 16), lambda i: (i, 0)),
            pl.BlockSpec((bn, 16), lambda i: (i, 0)),
            pl.BlockSpec((bn, 16), lambda i: (i, 0)),
            pl.BlockSpec((8, 16), lambda i: (0, 0)),
            pl.BlockSpec((8, 16), lambda i: (0, 0)),
        ],
        out_shape=[
            jax.ShapeDtypeStruct((NPAD, 16), jnp.float32),
            jax.ShapeDtypeStruct((NPAD, 16), jnp.float32),
            jax.ShapeDtypeStruct((NPAD, 16), jnp.float32),
            jax.ShapeDtypeStruct((8, 16), jnp.float32),
            jax.ShapeDtypeStruct((8, 16), jnp.float32),
        ],
    )(out2c, b1b, w2p, ms2, md2)


def _tc4_body(o_ref, b2_ref, out_ref):
    t = o_ref[0] + o_ref[1] + b2_ref[0:1, :]
    lane = lax.broadcasted_iota(jnp.int32, t.shape, 1)
    valid = lane < 7
    tm = jnp.where(valid, t, -3e38)
    m = jnp.max(tm, axis=1, keepdims=True)
    s = jnp.sum(jnp.where(valid, jnp.exp(t - m), 0.0), axis=1, keepdims=True)
    out_ref[...] = t - m - jnp.log(s)


def _tc4(out2c, b2b):
    bn = 512
    return pl.pallas_call(
        _tc4_body,
        grid=(NPAD // bn,),
        in_specs=[
            pl.BlockSpec((2, bn, 16), lambda i: (0, i, 0)),
            pl.BlockSpec((8, 16), lambda i: (0, 0)),
        ],
        out_specs=pl.BlockSpec((bn, 16), lambda i: (i, 0)),
        out_shape=jax.ShapeDtypeStruct((NPAD, 16), jnp.float32),
    )(out2c, b2b)


_MESH = plsc.VectorSubcoreMesh(core_axis_name="c", subcore_axis_name="s")
_SC_PARAMS = pltpu.CompilerParams(use_tc_tiling_on_sc=False)


@functools.partial(
    pl.kernel,
    mesh=_MESH,
    compiler_params=_SC_PARAMS,
    out_type=[
        jax.ShapeDtypeStruct((EP, 16), jnp.float32),
        jax.ShapeDtypeStruct((2, NPAD, 16), jnp.float32),
    ],
    scratch_types=[
        pltpu.VMEM((NCHUNKS, KC), jnp.int32),
        pltpu.VMEM((NCHUNKS, KC), jnp.int32),
        pltpu.VMEM((NB, KC, 16), jnp.float32),
        pltpu.VMEM((NB, KC, 16), jnp.float32),
        pltpu.VMEM((NB, KC, 16), jnp.float32),
        pltpu.VMEM((16,), jnp.float32),
        pltpu.VMEM((16,), jnp.float32),
        pltpu.VMEM_SHARED((NPAD, 16), jnp.float32),
        pltpu.SemaphoreType.DMA((NB,)),
        pltpu.SemaphoreType.DMA((NB,)),
        pltpu.SemaphoreType.DMA((NB,)),
    ],
)
def _sc_pass_a(src_hbm, dst_hbm, asrc_hbm, adst_hbm, maxs_hbm, maxd_hbm,
               zeros_hbm, ex_hbm, denom_hbm,
               src_all, dst_all, arows, drows, exbuf, m1, m2, denom_sp,
               gsem, osem, esem):
    cid = lax.axis_index("c")
    sid = lax.axis_index("s")
    wid = cid * 16 + sid
    row0 = wid * NCHUNKS

    pltpu.sync_copy(zeros_hbm.at[pl.ds(sid * ROWS_PER_TILE, ROWS_PER_TILE)],
                    denom_sp.at[pl.ds(sid * ROWS_PER_TILE, ROWS_PER_TILE)])
    pltpu.sync_copy(maxs_hbm.at[0], m1)
    pltpu.sync_copy(maxd_hbm.at[0], m2)
    pltpu.sync_copy(src_hbm.at[pl.ds(row0, NCHUNKS)], src_all)
    pltpu.sync_copy(dst_hbm.at[pl.ds(row0, NCHUNKS)], dst_all)
    sub = _lrelu(m1[...] + m2[...])
    plsc.subcore_barrier()

    def superiter(t, carry):
        gc = []
        for b in range(NB):
            c = t * NB + b
            gc.append((
                pltpu.async_copy(asrc_hbm.at[src_all.at[c]], arows.at[b],
                                 gsem.at[b]),
                pltpu.async_copy(adst_hbm.at[dst_all.at[c]], drows.at[b],
                                 gsem.at[b]),
            ))
        oc = []
        for b in range(NB):
            c = t * NB + b
            gc[b][0].wait()
            gc[b][1].wait()

            @plsc.parallel_loop(0, KC, unroll=8)
            def _edges(e, _b=b):
                v = _lrelu(arows[_b, e, :] + drows[_b, e, :])
                exbuf[_b, e, :] = jnp.exp(v - sub)

            base = pl.multiple_of(wid * EPW + c * KC, 8)
            oc.append((
                pltpu.async_copy(exbuf.at[b], denom_sp.at[dst_all.at[c]],
                                 osem.at[b], add=True),
                pltpu.async_copy(exbuf.at[b], ex_hbm.at[pl.ds(base, KC)],
                                 esem.at[b]),
            ))
        for b in range(NB):
            oc[b][0].wait()
            oc[b][1].wait()
        return carry

    lax.fori_loop(0, SUPERS, superiter, 0)
    plsc.subcore_barrier()
    pltpu.sync_copy(denom_sp.at[pl.ds(sid * ROWS_PER_TILE, ROWS_PER_TILE)],
                    denom_hbm.at[cid, pl.ds(sid * ROWS_PER_TILE, ROWS_PER_TILE)])


def _make_sc_pass_b(width):
    nheads = width // 16

    unroll = 8 if nheads == 1 else 2
    nb = 2 if width == 128 else 4
    supers = NCHUNKS // nb

    @functools.partial(
        pl.kernel,
        mesh=_MESH,
        compiler_params=_SC_PARAMS,
        out_type=jax.ShapeDtypeStruct((2, NPAD, width), jnp.float32),
        scratch_types=[
            pltpu.VMEM((nb, KC), jnp.int32),
            pltpu.VMEM((nb, KC), jnp.int32),
            pltpu.VMEM((nb, KC, width), jnp.float32),
            pltpu.VMEM((nb, KC, 16), jnp.float32),
            pltpu.VMEM((nb, KC, 16), jnp.float32),
            pltpu.VMEM_SHARED((NPAD, width), jnp.float32),
            pltpu.SemaphoreType.DMA((nb,)),
            pltpu.SemaphoreType.DMA((nb,)),
        ],
    )
    def _sc_pass_b(src_hbm, dst_hbm, ex_hbm, inv_hbm, h_hbm, zeros_hbm,
                   out_hbm, src_idx, dst_idx, hrows, exrows, invrows, out_sp,
                   gsem, osem):
        cid = lax.axis_index("c")
        sid = lax.axis_index("s")
        wid = cid * 16 + sid
        row0 = wid * NCHUNKS

        pltpu.sync_copy(zeros_hbm.at[pl.ds(sid * ROWS_PER_TILE, ROWS_PER_TILE)],
                        out_sp.at[pl.ds(sid * ROWS_PER_TILE, ROWS_PER_TILE)])
        plsc.subcore_barrier()

        def superiter(t, carry):
            crow = row0 + t * nb
            pltpu.sync_copy(src_hbm.at[pl.ds(crow, nb)], src_idx)
            pltpu.sync_copy(dst_hbm.at[pl.ds(crow, nb)], dst_idx)
            gc = []
            for b in range(nb):
                c = t * nb + b
                base = pl.multiple_of(wid * EPW + c * KC, 8)
                gc.append((
                    pltpu.async_copy(h_hbm.at[src_idx.at[b]], hrows.at[b],
                                     gsem.at[b]),
                    pltpu.async_copy(inv_hbm.at[dst_idx.at[b]], invrows.at[b],
                                     gsem.at[b]),
                    pltpu.async_copy(ex_hbm.at[pl.ds(base, KC)], exrows.at[b],
                                     gsem.at[b]),
                ))
            oc = []
            for b in range(nb):
                for g in gc[b]:
                    g.wait()

                @plsc.parallel_loop(0, KC, unroll=unroll)
                def _edges(e, _b=b):
                    coef = exrows[_b, e, :] * invrows[_b, e, :]
                    for hd in range(nheads):
                        hv = hrows[_b, e, pl.ds(hd * 16, 16)]
                        hrows[_b, e, pl.ds(hd * 16, 16)] = hv * _splat(coef, hd)

                oc.append(
                    pltpu.async_copy(hrows.at[b], out_sp.at[dst_idx.at[b]],
                                     osem.at[b], add=True))
            for b in range(nb):
                oc[b].wait()
            return carry

        lax.fori_loop(0, supers, superiter, 0)
        plsc.subcore_barrier()
        pltpu.sync_copy(out_sp.at[pl.ds(sid * ROWS_PER_TILE, ROWS_PER_TILE)],
                        out_hbm.at[cid, pl.ds(sid * ROWS_PER_TILE, ROWS_PER_TILE)])

    return _sc_pass_b


_sc_pass_b128 = _make_sc_pass_b(128)
_sc_pass_b16 = _make_sc_pass_b(16)


def kernel(x, edge_index, W1, a_src1, a_dst1, b1, W2, a_src2, a_dst2, b2):
    n = N_NODES
    loop = jnp.arange(n, dtype=jnp.int32)
    pad_e = EP - (E_RAW + n)
    padv = jnp.full((pad_e,), n, dtype=jnp.int32)
    srcp = jnp.concatenate([edge_index[0], loop, padv]).reshape(EP // KC, KC)
    dstp = jnp.concatenate([edge_index[1], loop, padv]).reshape(EP // KC, KC)

    xp = jnp.zeros((NPAD, 128), jnp.float32).at[:n].set(x)

    hd_ids = jnp.repeat(jnp.arange(8), 16)
    sel = (hd_ids[:, None] == jnp.arange(8)[None, :])
    ms1 = jnp.pad(jnp.where(sel, a_src1.reshape(-1)[:, None], 0.0), ((0, 0), (0, 8)))
    md1 = jnp.pad(jnp.where(sel, a_dst1.reshape(-1)[:, None], 0.0), ((0, 0), (0, 8)))

    w2p = jnp.pad(W2, ((0, 0), (0, 16 - W2.shape[1])))
    ms2 = jnp.pad(W2 @ a_src2.T, ((0, 0), (0, 15)))
    md2 = jnp.pad(W2 @ a_dst2.T, ((0, 0), (0, 15)))
    b1b = jnp.broadcast_to(b1[None, :], (8, 128))
    b2b = jnp.broadcast_to(jnp.pad(b2, (0, 16 - b2.shape[0]))[None, :], (8, 16))
    zeros128 = jnp.zeros((NPAD, 128), jnp.float32)
    zeros16 = jnp.zeros((NPAD, 16), jnp.float32)

    h1, as1, ad1, maxs1, maxd1 = _tc1(xp, W1, ms1, md1)
    ex1, denom1 = _sc_pass_a(srcp, dstp, as1, ad1, maxs1, maxd1, zeros16)
    inv1 = _tc_inv(denom1)
    out1 = _sc_pass_b128(srcp, dstp, ex1, inv1, h1, zeros128)

    h2, as2, ad2, maxs2, maxd2 = _tc3(out1, b1b, w2p, ms2, md2)
    ex2, denom2 = _sc_pass_a(srcp, dstp, as2, ad2, maxs2, maxd2, zeros16)
    inv2 = _tc_inv(denom2)
    out2 = _sc_pass_b16(srcp, dstp, ex2, inv2, h2, zeros16)

    res = _tc4(out2, b2b)
    return res[:n, :7]


# speedup vs baseline: 75.6172x; 1.0965x over previous
import functools

import jax
import jax.numpy as jnp
from jax import lax
from jax.experimental import pallas as pl
from jax.experimental.pallas import tpu as pltpu
from jax.experimental.pallas import tpu_sc as plsc

N_NODES = 10000
NPAD = 10240
E_RAW = 320000
EP = 331776
EPW = EP // 32
KC = 96
NCHUNKS = EPW // KC
NB = 4
SUPERS = NCHUNKS // NB
ROWS_PER_TILE = NPAD // 16

F1 = 128
F2 = 16


def _lrelu(v):
    return jnp.maximum(v, 0.2 * v)


def _splat(vec, i):
    return jnp.full((16,), vec[i], vec.dtype)


def _tc1_body(x_ref, w1_ref, ms_ref, md_ref, h_ref, as_ref, ad_ref,
              maxs_ref, maxd_ref):
    i = pl.program_id(0)
    h = jnp.dot(x_ref[...], w1_ref[...], preferred_element_type=jnp.float32)
    h_ref[...] = h
    a_s = jnp.dot(h, ms_ref[...], preferred_element_type=jnp.float32)
    a_d = jnp.dot(h, md_ref[...], preferred_element_type=jnp.float32)
    as_ref[...] = a_s
    ad_ref[...] = a_d

    @pl.when(i == 0)
    def _():
        maxs_ref[...] = jnp.full((8, 16), -3e38, jnp.float32)
        maxd_ref[...] = jnp.full((8, 16), -3e38, jnp.float32)

    bs = jnp.broadcast_to(jnp.max(a_s, axis=0, keepdims=True), (8, 16))
    bd = jnp.broadcast_to(jnp.max(a_d, axis=0, keepdims=True), (8, 16))
    maxs_ref[...] = jnp.maximum(maxs_ref[...], bs)
    maxd_ref[...] = jnp.maximum(maxd_ref[...], bd)


def _tc1(x, w1, ms, md):
    bn = 256
    grid = NPAD // bn
    return pl.pallas_call(
        _tc1_body,
        grid=(grid,),
        in_specs=[
            pl.BlockSpec((bn, 128), lambda i: (i, 0)),
            pl.BlockSpec((128, 128), lambda i: (0, 0)),
            pl.BlockSpec((128, 16), lambda i: (0, 0)),
            pl.BlockSpec((128, 16), lambda i: (0, 0)),
        ],
        out_specs=[
            pl.BlockSpec((bn, 128), lambda i: (i, 0)),
            pl.BlockSpec((bn, 16), lambda i: (i, 0)),
            pl.BlockSpec((bn, 16), lambda i: (i, 0)),
            pl.BlockSpec((8, 16), lambda i: (0, 0)),
            pl.BlockSpec((8, 16), lambda i: (0, 0)),
        ],
        out_shape=[
            jax.ShapeDtypeStruct((NPAD, 128), jnp.float32),
            jax.ShapeDtypeStruct((NPAD, 16), jnp.float32),
            jax.ShapeDtypeStruct((NPAD, 16), jnp.float32),
            jax.ShapeDtypeStruct((8, 16), jnp.float32),
            jax.ShapeDtypeStruct((8, 16), jnp.float32),
        ],
    )(x, w1, ms, md)


def _tc3_body(o_ref, b1_ref, w2p_ref, ms2_ref, md2_ref,
              h2_ref, as2_ref, ad2_ref, maxs_ref, maxd_ref):
    i = pl.program_id(0)
    t = o_ref[0] + o_ref[1] + b1_ref[0:1, :]
    h2pre = jnp.where(t > 0, t, jnp.exp(jnp.minimum(t, 0.0)) - 1.0)
    h2 = jnp.dot(h2pre, w2p_ref[...], preferred_element_type=jnp.float32)
    a_s = jnp.dot(h2pre, ms2_ref[...], preferred_element_type=jnp.float32)
    a_d = jnp.dot(h2pre, md2_ref[...], preferred_element_type=jnp.float32)
    h2_ref[...] = h2
    as2_ref[...] = a_s
    ad2_ref[...] = a_d

    @pl.when(i == 0)
    def _():
        maxs_ref[...] = jnp.full((8, 16), -3e38, jnp.float32)
        maxd_ref[...] = jnp.full((8, 16), -3e38, jnp.float32)

    bs = jnp.broadcast_to(jnp.max(a_s, axis=0, keepdims=True), (8, 16))
    bd = jnp.broadcast_to(jnp.max(a_d, axis=0, keepdims=True), (8, 16))
    maxs_ref[...] = jnp.maximum(maxs_ref[...], bs)
    maxd_ref[...] = jnp.maximum(maxd_ref[...], bd)


def _tc3(out2c, b1b, w2p, ms2, md2):
    bn = 256
    return pl.pallas_call(
        _tc3_body,
        grid=(NPAD // bn,),
        in_specs=[
            pl.BlockSpec((2, bn, 128), lambda i: (0, i, 0)),
            pl.BlockSpec((8, 128), lambda i: (0, 0)),
            pl.BlockSpec((128, 16), lambda i: (0, 0)),
            pl.BlockSpec((128, 16), lambda i: (0, 0)),
            pl.BlockSpec((128, 16), lambda i: (0, 0)),
        ],
        out_specs=[
            pl.BlockSpec((bn, 16), lambda i: (i, 0)),
            pl.BlockSpec((bn, 16), lambda i: (i, 0)),
            pl.BlockSpec((bn, 16), lambda i: (i, 0)),
            pl.BlockSpec((8, 16), lambda i: (0, 0)),
            pl.BlockSpec((8, 16), lambda i: (0, 0)),
        ],
        out_shape=[
            jax.ShapeDtypeStruct((NPAD, 16), jnp.float32),
            jax.ShapeDtypeStruct((NPAD, 16), jnp.float32),
            jax.ShapeDtypeStruct((NPAD, 16), jnp.float32),
            jax.ShapeDtypeStruct((8, 16), jnp.float32),
            jax.ShapeDtypeStruct((8, 16), jnp.float32),
        ],
    )(out2c, b1b, w2p, ms2, md2)


def _tc4_body(o_ref, b2_ref, out_ref):
    t = o_ref[0] + o_ref[1] + b2_ref[0:1, :]
    lane = lax.broadcasted_iota(jnp.int32, t.shape, 1)
    valid = lane < 7
    tm = jnp.where(valid, t, -3e38)
    m = jnp.max(tm, axis=1, keepdims=True)
    s = jnp.sum(jnp.where(valid, jnp.exp(t - m), 0.0), axis=1, keepdims=True)
    out_ref[...] = t - m - jnp.log(s)


def _tc4(out2c, b2b):
    bn = 512
    return pl.pallas_call(
        _tc4_body,
        grid=(NPAD // bn,),
        in_specs=[
            pl.BlockSpec((2, bn, 16), lambda i: (0, i, 0)),
            pl.BlockSpec((8, 16), lambda i: (0, 0)),
        ],
        out_specs=pl.BlockSpec((bn, 16), lambda i: (i, 0)),
        out_shape=jax.ShapeDtypeStruct((NPAD, 16), jnp.float32),
    )(out2c, b2b)


_MESH = plsc.VectorSubcoreMesh(core_axis_name="c", subcore_axis_name="s")
_SC_PARAMS = pltpu.CompilerParams(use_tc_tiling_on_sc=False)


@functools.partial(
    pl.kernel,
    mesh=_MESH,
    compiler_params=_SC_PARAMS,
    out_type=[
        jax.ShapeDtypeStruct((EP, 16), jnp.float32),
        jax.ShapeDtypeStruct((NPAD, 16), jnp.float32),
        jax.ShapeDtypeStruct((NPAD, 16), jnp.float32),
    ],
    scratch_types=[
        pltpu.VMEM((NCHUNKS, 2, KC), jnp.int32),
        pltpu.VMEM((NB, KC, 16), jnp.float32),
        pltpu.VMEM((NB, KC, 16), jnp.float32),
        pltpu.VMEM((NB, KC, 16), jnp.float32),
        pltpu.VMEM((16,), jnp.float32),
        pltpu.VMEM((16,), jnp.float32),
        pltpu.VMEM_SHARED((NPAD, 16), jnp.float32),
        pltpu.SemaphoreType.DMA((NB,)),
        pltpu.SemaphoreType.DMA((NB,)),
        pltpu.SemaphoreType.DMA((NB,)),
    ],
)
def _sc_pass_a(edge_hbm, asrc_hbm, adst_hbm, maxs_hbm, maxd_hbm,
               zeros_hbm, ex_hbm, den0_hbm, den1_hbm,
               edge_all, arows, drows, exbuf, m1, m2, denom_sp,
               gsem, osem, esem):
    cid = lax.axis_index("c")
    sid = lax.axis_index("s")
    wid = cid * 16 + sid
    row0 = wid * NCHUNKS

    pltpu.sync_copy(zeros_hbm.at[pl.ds(sid * ROWS_PER_TILE, ROWS_PER_TILE)],
                    denom_sp.at[pl.ds(sid * ROWS_PER_TILE, ROWS_PER_TILE)])
    pltpu.sync_copy(maxs_hbm.at[0], m1)
    pltpu.sync_copy(maxd_hbm.at[0], m2)
    pltpu.sync_copy(edge_hbm.at[pl.ds(row0, NCHUNKS)], edge_all)
    sub = _lrelu(m1[...] + m2[...])
    plsc.subcore_barrier()

    def superiter(t, carry):
        gc = []
        for b in range(NB):
            c = t * NB + b
            gc.append((
                pltpu.async_copy(asrc_hbm.at[edge_all.at[c, 0]], arows.at[b],
                                 gsem.at[b]),
                pltpu.async_copy(adst_hbm.at[edge_all.at[c, 1]], drows.at[b],
                                 gsem.at[b]),
            ))
        oc = []
        for b in range(NB):
            c = t * NB + b
            gc[b][0].wait()
            gc[b][1].wait()

            @plsc.parallel_loop(0, KC, unroll=8)
            def _edges(e, _b=b):
                v = _lrelu(arows[_b, e, :] + drows[_b, e, :])
                exbuf[_b, e, :] = jnp.exp(v - sub)

            base = pl.multiple_of(wid * EPW + c * KC, 8)
            oc.append((
                pltpu.async_copy(exbuf.at[b], denom_sp.at[edge_all.at[c, 1]],
                                 osem.at[b], add=True),
                pltpu.async_copy(exbuf.at[b], ex_hbm.at[pl.ds(base, KC)],
                                 esem.at[b]),
            ))
        for b in range(NB):
            oc[b][0].wait()
            oc[b][1].wait()
        return carry

    lax.fori_loop(0, SUPERS, superiter, 0)
    plsc.subcore_barrier()

    @pl.when(cid == 0)
    def _():
        pltpu.sync_copy(
            denom_sp.at[pl.ds(sid * ROWS_PER_TILE, ROWS_PER_TILE)],
            den0_hbm.at[pl.ds(sid * ROWS_PER_TILE, ROWS_PER_TILE)])

    @pl.when(cid == 1)
    def _():
        pltpu.sync_copy(
            denom_sp.at[pl.ds(sid * ROWS_PER_TILE, ROWS_PER_TILE)],
            den1_hbm.at[pl.ds(sid * ROWS_PER_TILE, ROWS_PER_TILE)])


def _make_sc_pass_b(width):
    nheads = width // 16

    unroll = 8 if nheads == 1 else 2
    nb = 2 if width == 128 else 6
    supers = NCHUNKS // nb

    @functools.partial(
        pl.kernel,
        mesh=_MESH,
        compiler_params=_SC_PARAMS,
        out_type=jax.ShapeDtypeStruct((2, NPAD, width), jnp.float32),
        scratch_types=[
            pltpu.VMEM((2, nb, 2, KC), jnp.int32),
            pltpu.VMEM((nb, KC, width), jnp.float32),
            pltpu.VMEM((nb, KC, 16), jnp.float32),
            pltpu.VMEM((nb, KC, 16), jnp.float32),
            pltpu.VMEM((nb, KC, 16), jnp.float32),
            pltpu.VMEM_SHARED((NPAD, width), jnp.float32),
            pltpu.SemaphoreType.DMA((nb,)),
            pltpu.SemaphoreType.DMA((nb,)),
            pltpu.SemaphoreType.DMA((2,)),
        ],
    )
    def _sc_pass_b(edge_hbm, ex_hbm, den0_hbm, den1_hbm, h_hbm, zeros_hbm,
                   out_hbm, eidx, hrows, exrows, d0rows, d1rows, out_sp,
                   gsem, osem, isem):
        cid = lax.axis_index("c")
        sid = lax.axis_index("s")
        wid = cid * 16 + sid
        row0 = wid * NCHUNKS

        pltpu.sync_copy(zeros_hbm.at[pl.ds(sid * ROWS_PER_TILE, ROWS_PER_TILE)],
                        out_sp.at[pl.ds(sid * ROWS_PER_TILE, ROWS_PER_TILE)])
        pltpu.sync_copy(edge_hbm.at[pl.ds(row0, nb)], eidx.at[0])
        plsc.subcore_barrier()

        def superiter(t, p):
            nxt = pltpu.async_copy(
                edge_hbm.at[pl.ds(row0 + (t + 1) * nb, nb)], eidx.at[1 - p],
                isem.at[1 - p])
            gc = []
            for b in range(nb):
                c = t * nb + b
                base = pl.multiple_of(wid * EPW + c * KC, 8)
                gc.append((
                    pltpu.async_copy(h_hbm.at[eidx.at[p, b, 0]], hrows.at[b],
                                     gsem.at[b]),
                    pltpu.async_copy(den0_hbm.at[eidx.at[p, b, 1]],
                                     d0rows.at[b], gsem.at[b]),
                    pltpu.async_copy(den1_hbm.at[eidx.at[p, b, 1]],
                                     d1rows.at[b], gsem.at[b]),
                    pltpu.async_copy(ex_hbm.at[pl.ds(base, KC)], exrows.at[b],
                                     gsem.at[b]),
                ))
            oc = []
            for b in range(nb):
                for g in gc[b]:
                    g.wait()

                @plsc.parallel_loop(0, KC, unroll=unroll)
                def _edges(e, _b=b):
                    den = d0rows[_b, e, :] + d1rows[_b, e, :] + 1e-16
                    coef = exrows[_b, e, :] / den
                    for hd in range(nheads):
                        hv = hrows[_b, e, pl.ds(hd * 16, 16)]
                        hrows[_b, e, pl.ds(hd * 16, 16)] = hv * _splat(coef, hd)

                oc.append(
                    pltpu.async_copy(hrows.at[b],
                                     out_sp.at[eidx.at[p, b, 1]],
                                     osem.at[b], add=True))
            for b in range(nb):
                oc[b].wait()
            nxt.wait()

        def pair(u, carry):
            superiter(2 * u, 0)
            superiter(2 * u + 1, 1)
            return carry

        lax.fori_loop(0, supers // 2, pair, 0)
        plsc.subcore_barrier()
        pltpu.sync_copy(out_sp.at[pl.ds(sid * ROWS_PER_TILE, ROWS_PER_TILE)],
                        out_hbm.at[cid, pl.ds(sid * ROWS_PER_TILE, ROWS_PER_TILE)])

    return _sc_pass_b


_sc_pass_b128 = _make_sc_pass_b(128)
_sc_pass_b16 = _make_sc_pass_b(16)


def kernel(x, edge_index, W1, a_src1, a_dst1, b1, W2, a_src2, a_dst2, b2):
    n = N_NODES
    loop = jnp.arange(n, dtype=jnp.int32)
    pad_e = EP - (E_RAW + n)
    padv = jnp.full((pad_e,), n, dtype=jnp.int32)
    srcp = jnp.concatenate([edge_index[0], loop, padv]).reshape(EP // KC, KC)
    dstp = jnp.concatenate([edge_index[1], loop, padv]).reshape(EP // KC, KC)
    edge2d = jnp.concatenate(
        [jnp.stack([srcp, dstp], axis=1),
         jnp.zeros((6, 2, KC), jnp.int32)], axis=0)

    xp = jnp.zeros((NPAD, 128), jnp.float32).at[:n].set(x)

    hd_ids = jnp.repeat(jnp.arange(8), 16)
    sel = (hd_ids[:, None] == jnp.arange(8)[None, :])
    ms1 = jnp.pad(jnp.where(sel, a_src1.reshape(-1)[:, None], 0.0), ((0, 0), (0, 8)))
    md1 = jnp.pad(jnp.where(sel, a_dst1.reshape(-1)[:, None], 0.0), ((0, 0), (0, 8)))

    w2p = jnp.pad(W2, ((0, 0), (0, 16 - W2.shape[1])))
    ms2 = jnp.pad(W2 @ a_src2.T, ((0, 0), (0, 15)))
    md2 = jnp.pad(W2 @ a_dst2.T, ((0, 0), (0, 15)))
    b1b = jnp.broadcast_to(b1[None, :], (8, 128))
    b2b = jnp.broadcast_to(jnp.pad(b2, (0, 16 - b2.shape[0]))[None, :], (8, 16))
    zeros128 = jnp.zeros((NPAD, 128), jnp.float32)
    zeros16 = jnp.zeros((NPAD, 16), jnp.float32)

    h1, as1, ad1, maxs1, maxd1 = _tc1(xp, W1, ms1, md1)
    ex1, dn0_1, dn1_1 = _sc_pass_a(edge2d, as1, ad1, maxs1, maxd1, zeros16)
    out1 = _sc_pass_b128(edge2d, ex1, dn0_1, dn1_1, h1, zeros128)

    h2, as2, ad2, maxs2, maxd2 = _tc3(out1, b1b, w2p, ms2, md2)
    ex2, dn0_2, dn1_2 = _sc_pass_a(edge2d, as2, ad2, maxs2, maxd2, zeros16)
    out2 = _sc_pass_b16(edge2d, ex2, dn0_2, dn1_2, h2, zeros16)

    res = _tc4(out2, b2b)
    return res[:n, :7]


# speedup vs baseline: 95.4162x; 1.2618x over previous
import functools

import jax
import jax.numpy as jnp
from jax import lax
from jax.experimental import pallas as pl
from jax.experimental.pallas import tpu as pltpu
from jax.experimental.pallas import tpu_sc as plsc

N_NODES = 10000
NPAD = 10240
E_RAW = 320000
EP = 331776
EPW = EP // 32
KC = 96
NCHUNKS = EPW // KC
NB = 6
SUPERS = NCHUNKS // NB
ROWS_PER_TILE = NPAD // 16

F1 = 128
F2 = 16


def _lrelu(v):
    return jnp.maximum(v, 0.2 * v)


def _splat(vec, i):
    return jnp.full((16,), vec[i], vec.dtype)


def _tc1_body(x_ref, w1_ref, ms_ref, md_ref, h_ref, as_ref, ad_ref,
              maxs_ref, maxd_ref):
    i = pl.program_id(0)
    h = jnp.dot(x_ref[...], w1_ref[...], preferred_element_type=jnp.float32)
    h_ref[...] = h
    a_s = jnp.dot(h, ms_ref[...], preferred_element_type=jnp.float32)
    a_d = jnp.dot(h, md_ref[...], preferred_element_type=jnp.float32)
    as_ref[...] = a_s
    ad_ref[...] = a_d

    @pl.when(i == 0)
    def _():
        maxs_ref[...] = jnp.full((8, 16), -3e38, jnp.float32)
        maxd_ref[...] = jnp.full((8, 16), -3e38, jnp.float32)

    bs = jnp.broadcast_to(jnp.max(a_s, axis=0, keepdims=True), (8, 16))
    bd = jnp.broadcast_to(jnp.max(a_d, axis=0, keepdims=True), (8, 16))
    maxs_ref[...] = jnp.maximum(maxs_ref[...], bs)
    maxd_ref[...] = jnp.maximum(maxd_ref[...], bd)


def _tc1(x, w1, ms, md):
    bn = 256
    grid = NPAD // bn
    return pl.pallas_call(
        _tc1_body,
        grid=(grid,),
        in_specs=[
            pl.BlockSpec((bn, 128), lambda i: (i, 0)),
            pl.BlockSpec((128, 128), lambda i: (0, 0)),
            pl.BlockSpec((128, 16), lambda i: (0, 0)),
            pl.BlockSpec((128, 16), lambda i: (0, 0)),
        ],
        out_specs=[
            pl.BlockSpec((bn, 128), lambda i: (i, 0)),
            pl.BlockSpec((bn, 16), lambda i: (i, 0)),
            pl.BlockSpec((bn, 16), lambda i: (i, 0)),
            pl.BlockSpec((8, 16), lambda i: (0, 0)),
            pl.BlockSpec((8, 16), lambda i: (0, 0)),
        ],
        out_shape=[
            jax.ShapeDtypeStruct((NPAD, 128), jnp.float32),
            jax.ShapeDtypeStruct((NPAD, 16), jnp.float32),
            jax.ShapeDtypeStruct((NPAD, 16), jnp.float32),
            jax.ShapeDtypeStruct((8, 16), jnp.float32),
            jax.ShapeDtypeStruct((8, 16), jnp.float32),
        ],
    )(x, w1, ms, md)


def _tc3_body(o_ref, b1_ref, w2p_ref, ms2_ref, md2_ref,
              h2_ref, as2_ref, ad2_ref, maxs_ref, maxd_ref):
    i = pl.program_id(0)
    t = o_ref[0] + o_ref[1] + b1_ref[0:1, :]
    h2pre = jnp.where(t > 0, t, jnp.exp(jnp.minimum(t, 0.0)) - 1.0)
    h2 = jnp.dot(h2pre, w2p_ref[...], preferred_element_type=jnp.float32)
    a_s = jnp.dot(h2pre, ms2_ref[...], preferred_element_type=jnp.float32)
    a_d = jnp.dot(h2pre, md2_ref[...], preferred_element_type=jnp.float32)
    h2_ref[...] = h2
    as2_ref[...] = a_s
    ad2_ref[...] = a_d

    @pl.when(i == 0)
    def _():
        maxs_ref[...] = jnp.full((8, 16), -3e38, jnp.float32)
        maxd_ref[...] = jnp.full((8, 16), -3e38, jnp.float32)

    bs = jnp.broadcast_to(jnp.max(a_s, axis=0, keepdims=True), (8, 16))
    bd = jnp.broadcast_to(jnp.max(a_d, axis=0, keepdims=True), (8, 16))
    maxs_ref[...] = jnp.maximum(maxs_ref[...], bs)
    maxd_ref[...] = jnp.maximum(maxd_ref[...], bd)


def _tc3(out2c, b1b, w2p, ms2, md2):
    bn = 256
    return pl.pallas_call(
        _tc3_body,
        grid=(NPAD // bn,),
        in_specs=[
            pl.BlockSpec((2, bn, 128), lambda i: (0, i, 0)),
            pl.BlockSpec((8, 128), lambda i: (0, 0)),
            pl.BlockSpec((128, 16), lambda i: (0, 0)),
            pl.BlockSpec((128, 16), lambda i: (0, 0)),
            pl.BlockSpec((128, 16), lambda i: (0, 0)),
        ],
        out_specs=[
            pl.BlockSpec((bn, 16), lambda i: (i, 0)),
            pl.BlockSpec((bn, 16), lambda i: (i, 0)),
            pl.BlockSpec((bn, 16), lambda i: (i, 0)),
            pl.BlockSpec((8, 16), lambda i: (0, 0)),
            pl.BlockSpec((8, 16), lambda i: (0, 0)),
        ],
        out_shape=[
            jax.ShapeDtypeStruct((NPAD, 16), jnp.float32),
            jax.ShapeDtypeStruct((NPAD, 16), jnp.float32),
            jax.ShapeDtypeStruct((NPAD, 16), jnp.float32),
            jax.ShapeDtypeStruct((8, 16), jnp.float32),
            jax.ShapeDtypeStruct((8, 16), jnp.float32),
        ],
    )(out2c, b1b, w2p, ms2, md2)


def _tc4_body(o_ref, b2_ref, out_ref):
    t = o_ref[0] + o_ref[1] + b2_ref[0:1, :]
    lane = lax.broadcasted_iota(jnp.int32, t.shape, 1)
    valid = lane < 7
    tm = jnp.where(valid, t, -3e38)
    m = jnp.max(tm, axis=1, keepdims=True)
    s = jnp.sum(jnp.where(valid, jnp.exp(t - m), 0.0), axis=1, keepdims=True)
    out_ref[...] = t - m - jnp.log(s)


def _tc4(out2c, b2b):
    bn = 512
    return pl.pallas_call(
        _tc4_body,
        grid=(NPAD // bn,),
        in_specs=[
            pl.BlockSpec((2, bn, 16), lambda i: (0, i, 0)),
            pl.BlockSpec((8, 16), lambda i: (0, 0)),
        ],
        out_specs=pl.BlockSpec((bn, 16), lambda i: (i, 0)),
        out_shape=jax.ShapeDtypeStruct((NPAD, 16), jnp.float32),
    )(out2c, b2b)


_MESH = plsc.VectorSubcoreMesh(core_axis_name="c", subcore_axis_name="s")
_SC_PARAMS = pltpu.CompilerParams(use_tc_tiling_on_sc=False)


@functools.partial(
    pl.kernel,
    mesh=_MESH,
    compiler_params=_SC_PARAMS,
    out_type=[
        jax.ShapeDtypeStruct((EP, 16), jnp.float32),
        jax.ShapeDtypeStruct((NPAD, 16), jnp.float32),
        jax.ShapeDtypeStruct((NPAD, 16), jnp.float32),
    ],
    scratch_types=[
        pltpu.VMEM((NCHUNKS, 2, KC), jnp.int32),
        pltpu.VMEM((NB, KC, 16), jnp.float32),
        pltpu.VMEM((NB, KC, 16), jnp.float32),
        pltpu.VMEM((NB, KC, 16), jnp.float32),
        pltpu.VMEM((16,), jnp.float32),
        pltpu.VMEM((16,), jnp.float32),
        pltpu.VMEM_SHARED((NPAD, 16), jnp.float32),
        pltpu.SemaphoreType.DMA((NB,)),
        pltpu.SemaphoreType.DMA((NB,)),
        pltpu.SemaphoreType.DMA((NB,)),
    ],
)
def _sc_pass_a(edge_hbm, asrc_hbm, adst_hbm, maxs_hbm, maxd_hbm,
               zeros_hbm, ex_hbm, den0_hbm, den1_hbm,
               edge_all, arows, drows, exbuf, m1, m2, denom_sp,
               gsem, osem, esem):
    cid = lax.axis_index("c")
    sid = lax.axis_index("s")
    wid = cid * 16 + sid
    row0 = wid * NCHUNKS

    pltpu.sync_copy(zeros_hbm.at[pl.ds(sid * ROWS_PER_TILE, ROWS_PER_TILE)],
                    denom_sp.at[pl.ds(sid * ROWS_PER_TILE, ROWS_PER_TILE)])
    pltpu.sync_copy(maxs_hbm.at[0], m1)
    pltpu.sync_copy(maxd_hbm.at[0], m2)
    pltpu.sync_copy(edge_hbm.at[pl.ds(row0, NCHUNKS)], edge_all)
    sub = _lrelu(m1[...] + m2[...])
    plsc.subcore_barrier()

    def superiter(t, carry):
        gc = []
        for b in range(NB):
            c = t * NB + b
            gc.append((
                pltpu.async_copy(asrc_hbm.at[edge_all.at[c, 0]], arows.at[b],
                                 gsem.at[b]),
                pltpu.async_copy(adst_hbm.at[edge_all.at[c, 1]], drows.at[b],
                                 gsem.at[b]),
            ))
        oc = []
        for b in range(NB):
            c = t * NB + b
            gc[b][0].wait()
            gc[b][1].wait()

            @plsc.parallel_loop(0, KC, unroll=8)
            def _edges(e, _b=b):
                v = _lrelu(arows[_b, e, :] + drows[_b, e, :])
                exbuf[_b, e, :] = jnp.exp(v - sub)

            base = pl.multiple_of(wid * EPW + c * KC, 8)
            oc.append((
                pltpu.async_copy(exbuf.at[b], denom_sp.at[edge_all.at[c, 1]],
                                 osem.at[b], add=True),
                pltpu.async_copy(exbuf.at[b], ex_hbm.at[pl.ds(base, KC)],
                                 esem.at[b]),
            ))
        for b in range(NB):
            oc[b][0].wait()
            oc[b][1].wait()
        return carry

    lax.fori_loop(0, SUPERS, superiter, 0)
    plsc.subcore_barrier()

    @pl.when(cid == 0)
    def _():
        pltpu.sync_copy(
            denom_sp.at[pl.ds(sid * ROWS_PER_TILE, ROWS_PER_TILE)],
            den0_hbm.at[pl.ds(sid * ROWS_PER_TILE, ROWS_PER_TILE)])

    @pl.when(cid == 1)
    def _():
        pltpu.sync_copy(
            denom_sp.at[pl.ds(sid * ROWS_PER_TILE, ROWS_PER_TILE)],
            den1_hbm.at[pl.ds(sid * ROWS_PER_TILE, ROWS_PER_TILE)])


def _make_sc_pass_b(width):
    nheads = width // 16

    unroll = 8 if nheads == 1 else 2
    nb = 2 if width == 128 else 6
    supers = NCHUNKS // nb

    @functools.partial(
        pl.kernel,
        mesh=_MESH,
        compiler_params=_SC_PARAMS,
        out_type=jax.ShapeDtypeStruct((2, NPAD, width), jnp.float32),
        scratch_types=[
            pltpu.VMEM((2, nb, 2, KC), jnp.int32),
            pltpu.VMEM((nb, KC, width), jnp.float32),
            pltpu.VMEM((nb, KC, 16), jnp.float32),
            pltpu.VMEM((nb, KC, 16), jnp.float32),
            pltpu.VMEM((nb, KC, 16), jnp.float32),
            pltpu.VMEM_SHARED((NPAD, width), jnp.float32),
            pltpu.SemaphoreType.DMA((nb,)),
            pltpu.SemaphoreType.DMA((nb,)),
            pltpu.SemaphoreType.DMA((2,)),
        ],
    )
    def _sc_pass_b(edge_hbm, ex_hbm, den0_hbm, den1_hbm, h_hbm, zeros_hbm,
                   out_hbm, eidx, hrows, exrows, d0rows, d1rows, out_sp,
                   gsem, osem, isem):
        cid = lax.axis_index("c")
        sid = lax.axis_index("s")
        wid = cid * 16 + sid
        row0 = wid * NCHUNKS

        pltpu.sync_copy(zeros_hbm.at[pl.ds(sid * ROWS_PER_TILE, ROWS_PER_TILE)],
                        out_sp.at[pl.ds(sid * ROWS_PER_TILE, ROWS_PER_TILE)])
        pltpu.sync_copy(edge_hbm.at[pl.ds(row0, nb)], eidx.at[0])
        plsc.subcore_barrier()

        def superiter(t, p):
            nxt = pltpu.async_copy(
                edge_hbm.at[pl.ds(row0 + (t + 1) * nb, nb)], eidx.at[1 - p],
                isem.at[1 - p])
            gc = []
            for b in range(nb):
                c = t * nb + b
                base = pl.multiple_of(wid * EPW + c * KC, 8)
                gc.append((
                    pltpu.async_copy(h_hbm.at[eidx.at[p, b, 0]], hrows.at[b],
                                     gsem.at[b]),
                    pltpu.async_copy(den0_hbm.at[eidx.at[p, b, 1]],
                                     d0rows.at[b], gsem.at[b]),
                    pltpu.async_copy(den1_hbm.at[eidx.at[p, b, 1]],
                                     d1rows.at[b], gsem.at[b]),
                    pltpu.async_copy(ex_hbm.at[pl.ds(base, KC)], exrows.at[b],
                                     gsem.at[b]),
                ))
            oc = []
            for b in range(nb):
                for g in gc[b]:
                    g.wait()

                @plsc.parallel_loop(0, KC, unroll=unroll)
                def _edges(e, _b=b):
                    den = d0rows[_b, e, :] + d1rows[_b, e, :] + 1e-16
                    coef = exrows[_b, e, :] / den
                    for hd in range(nheads):
                        hv = hrows[_b, e, pl.ds(hd * 16, 16)]
                        hrows[_b, e, pl.ds(hd * 16, 16)] = hv * _splat(coef, hd)

                oc.append(
                    pltpu.async_copy(hrows.at[b],
                                     out_sp.at[eidx.at[p, b, 1]],
                                     osem.at[b], add=True))
            for b in range(nb):
                oc[b].wait()
            nxt.wait()

        def pair(u, carry):
            superiter(2 * u, 0)
            superiter(2 * u + 1, 1)
            return carry

        lax.fori_loop(0, supers // 2, pair, 0)
        plsc.subcore_barrier()
        pltpu.sync_copy(out_sp.at[pl.ds(sid * ROWS_PER_TILE, ROWS_PER_TILE)],
                        out_hbm.at[cid, pl.ds(sid * ROWS_PER_TILE, ROWS_PER_TILE)])

    return _sc_pass_b


_sc_pass_b128 = _make_sc_pass_b(128)
_sc_pass_b16 = _make_sc_pass_b(16)


def kernel(x, edge_index, W1, a_src1, a_dst1, b1, W2, a_src2, a_dst2, b2):
    n = N_NODES
    loop = jnp.arange(n, dtype=jnp.int32)
    pad_e = EP - (E_RAW + n)
    padv = n + (jnp.arange(pad_e, dtype=jnp.int32) % (NPAD - n))
    srcp = jnp.concatenate([edge_index[0], loop, padv]).reshape(EP // KC, KC)
    dstp = jnp.concatenate([edge_index[1], loop, padv]).reshape(EP // KC, KC)
    edge2d = jnp.concatenate(
        [jnp.stack([srcp, dstp], axis=1),
         jnp.zeros((6, 2, KC), jnp.int32)], axis=0)

    xp = jnp.zeros((NPAD, 128), jnp.float32).at[:n].set(x)

    hd_ids = jnp.repeat(jnp.arange(8), 16)
    sel = (hd_ids[:, None] == jnp.arange(8)[None, :])
    ms1 = jnp.pad(jnp.where(sel, a_src1.reshape(-1)[:, None], 0.0), ((0, 0), (0, 8)))
    md1 = jnp.pad(jnp.where(sel, a_dst1.reshape(-1)[:, None], 0.0), ((0, 0), (0, 8)))

    w2p = jnp.pad(W2, ((0, 0), (0, 16 - W2.shape[1])))
    ms2 = jnp.pad(W2 @ a_src2.T, ((0, 0), (0, 15)))
    md2 = jnp.pad(W2 @ a_dst2.T, ((0, 0), (0, 15)))
    b1b = jnp.broadcast_to(b1[None, :], (8, 128))
    b2b = jnp.broadcast_to(jnp.pad(b2, (0, 16 - b2.shape[0]))[None, :], (8, 16))
    zeros128 = jnp.zeros((NPAD, 128), jnp.float32)
    zeros16 = jnp.zeros((NPAD, 16), jnp.float32)

    h1, as1, ad1, maxs1, maxd1 = _tc1(xp, W1, ms1, md1)
    ex1, dn0_1, dn1_1 = _sc_pass_a(edge2d, as1, ad1, maxs1, maxd1, zeros16)
    out1 = _sc_pass_b128(edge2d, ex1, dn0_1, dn1_1, h1, zeros128)

    h2, as2, ad2, maxs2, maxd2 = _tc3(out1, b1b, w2p, ms2, md2)
    ex2, dn0_2, dn1_2 = _sc_pass_a(edge2d, as2, ad2, maxs2, maxd2, zeros16)
    out2 = _sc_pass_b16(edge2d, ex2, dn0_2, dn1_2, h2, zeros16)

    res = _tc4(out2, b2b)
    return res[:n, :7]


# speedup vs baseline: 97.4956x; 1.0218x over previous
import functools

import jax
import jax.numpy as jnp
from jax import lax
from jax.experimental import pallas as pl
from jax.experimental.pallas import tpu as pltpu
from jax.experimental.pallas import tpu_sc as plsc

N_NODES = 10000
NPAD = 10240
E_RAW = 320000
EP = 331776
EPW = EP // 32
KC = 96
NCHUNKS = EPW // KC
NB = 6
SUPERS = NCHUNKS // NB
ROWS_PER_TILE = NPAD // 16

F1 = 128
F2 = 16


def _lrelu(v):
    return jnp.maximum(v, 0.2 * v)


def _splat(vec, i):
    return jnp.full((16,), vec[i], vec.dtype)


def _tc1_body(x_ref, w1_ref, ms_ref, md_ref, h_ref, as_ref, ad_ref,
              maxs_ref, maxd_ref):
    i = pl.program_id(0)
    h = jnp.dot(x_ref[...], w1_ref[...], preferred_element_type=jnp.float32)
    h_ref[...] = h
    a_s = jnp.dot(h, ms_ref[...], preferred_element_type=jnp.float32)
    a_d = jnp.dot(h, md_ref[...], preferred_element_type=jnp.float32)
    as_ref[...] = a_s
    ad_ref[...] = a_d

    @pl.when(i == 0)
    def _():
        maxs_ref[...] = jnp.full((8, 16), -3e38, jnp.float32)
        maxd_ref[...] = jnp.full((8, 16), -3e38, jnp.float32)

    bs = jnp.broadcast_to(jnp.max(a_s, axis=0, keepdims=True), (8, 16))
    bd = jnp.broadcast_to(jnp.max(a_d, axis=0, keepdims=True), (8, 16))
    maxs_ref[...] = jnp.maximum(maxs_ref[...], bs)
    maxd_ref[...] = jnp.maximum(maxd_ref[...], bd)


def _tc1(x, w1, ms, md):
    bn = 256
    grid = NPAD // bn
    return pl.pallas_call(
        _tc1_body,
        grid=(grid,),
        in_specs=[
            pl.BlockSpec((bn, 128), lambda i: (i, 0)),
            pl.BlockSpec((128, 128), lambda i: (0, 0)),
            pl.BlockSpec((128, 16), lambda i: (0, 0)),
            pl.BlockSpec((128, 16), lambda i: (0, 0)),
        ],
        out_specs=[
            pl.BlockSpec((bn, 128), lambda i: (i, 0)),
            pl.BlockSpec((bn, 16), lambda i: (i, 0)),
            pl.BlockSpec((bn, 16), lambda i: (i, 0)),
            pl.BlockSpec((8, 16), lambda i: (0, 0)),
            pl.BlockSpec((8, 16), lambda i: (0, 0)),
        ],
        out_shape=[
            jax.ShapeDtypeStruct((NPAD, 128), jnp.float32),
            jax.ShapeDtypeStruct((NPAD, 16), jnp.float32),
            jax.ShapeDtypeStruct((NPAD, 16), jnp.float32),
            jax.ShapeDtypeStruct((8, 16), jnp.float32),
            jax.ShapeDtypeStruct((8, 16), jnp.float32),
        ],
    )(x, w1, ms, md)


def _tc3_body(o_ref, b1_ref, w2p_ref, ms2_ref, md2_ref,
              h2_ref, as2_ref, ad2_ref, maxs_ref, maxd_ref):
    i = pl.program_id(0)
    t = o_ref[0] + o_ref[1] + b1_ref[0:1, :]
    h2pre = jnp.where(t > 0, t, jnp.exp(jnp.minimum(t, 0.0)) - 1.0)
    h2 = jnp.dot(h2pre, w2p_ref[...], preferred_element_type=jnp.float32)
    a_s = jnp.dot(h2pre, ms2_ref[...], preferred_element_type=jnp.float32)
    a_d = jnp.dot(h2pre, md2_ref[...], preferred_element_type=jnp.float32)
    h2_ref[...] = h2
    as2_ref[...] = a_s
    ad2_ref[...] = a_d

    @pl.when(i == 0)
    def _():
        maxs_ref[...] = jnp.full((8, 16), -3e38, jnp.float32)
        maxd_ref[...] = jnp.full((8, 16), -3e38, jnp.float32)

    bs = jnp.broadcast_to(jnp.max(a_s, axis=0, keepdims=True), (8, 16))
    bd = jnp.broadcast_to(jnp.max(a_d, axis=0, keepdims=True), (8, 16))
    maxs_ref[...] = jnp.maximum(maxs_ref[...], bs)
    maxd_ref[...] = jnp.maximum(maxd_ref[...], bd)


def _tc3(out2c, b1b, w2p, ms2, md2):
    bn = 256
    return pl.pallas_call(
        _tc3_body,
        grid=(NPAD // bn,),
        in_specs=[
            pl.BlockSpec((2, bn, 128), lambda i: (0, i, 0)),
            pl.BlockSpec((8, 128), lambda i: (0, 0)),
            pl.BlockSpec((128, 16), lambda i: (0, 0)),
            pl.BlockSpec((128, 16), lambda i: (0, 0)),
            pl.BlockSpec((128, 16), lambda i: (0, 0)),
        ],
        out_specs=[
            pl.BlockSpec((bn, 16), lambda i: (i, 0)),
            pl.BlockSpec((bn, 16), lambda i: (i, 0)),
            pl.BlockSpec((bn, 16), lambda i: (i, 0)),
            pl.BlockSpec((8, 16), lambda i: (0, 0)),
            pl.BlockSpec((8, 16), lambda i: (0, 0)),
        ],
        out_shape=[
            jax.ShapeDtypeStruct((NPAD, 16), jnp.float32),
            jax.ShapeDtypeStruct((NPAD, 16), jnp.float32),
            jax.ShapeDtypeStruct((NPAD, 16), jnp.float32),
            jax.ShapeDtypeStruct((8, 16), jnp.float32),
            jax.ShapeDtypeStruct((8, 16), jnp.float32),
        ],
    )(out2c, b1b, w2p, ms2, md2)


def _tc4_body(o_ref, b2_ref, out_ref):
    t = o_ref[0] + o_ref[1] + b2_ref[0:1, :]
    lane = lax.broadcasted_iota(jnp.int32, t.shape, 1)
    valid = lane < 7
    tm = jnp.where(valid, t, -3e38)
    m = jnp.max(tm, axis=1, keepdims=True)
    s = jnp.sum(jnp.where(valid, jnp.exp(t - m), 0.0), axis=1, keepdims=True)
    out_ref[...] = t - m - jnp.log(s)


def _tc4(out2c, b2b):
    bn = 512
    return pl.pallas_call(
        _tc4_body,
        grid=(NPAD // bn,),
        in_specs=[
            pl.BlockSpec((2, bn, 16), lambda i: (0, i, 0)),
            pl.BlockSpec((8, 16), lambda i: (0, 0)),
        ],
        out_specs=pl.BlockSpec((bn, 16), lambda i: (i, 0)),
        out_shape=jax.ShapeDtypeStruct((NPAD, 16), jnp.float32),
    )(out2c, b2b)


_MESH = plsc.VectorSubcoreMesh(core_axis_name="c", subcore_axis_name="s")
_SC_PARAMS = pltpu.CompilerParams(use_tc_tiling_on_sc=False)


@functools.partial(
    pl.kernel,
    mesh=_MESH,
    compiler_params=_SC_PARAMS,
    out_type=[
        jax.ShapeDtypeStruct((EP, 16), jnp.float32),
        jax.ShapeDtypeStruct((NPAD, 16), jnp.float32),
        jax.ShapeDtypeStruct((NPAD, 16), jnp.float32),
    ],
    scratch_types=[
        pltpu.VMEM((NCHUNKS, 2, KC), jnp.int32),
        pltpu.VMEM((NB, KC, 16), jnp.float32),
        pltpu.VMEM((NB, KC, 16), jnp.float32),
        pltpu.VMEM((NB, KC, 16), jnp.float32),
        pltpu.VMEM((16,), jnp.float32),
        pltpu.VMEM((16,), jnp.float32),
        pltpu.VMEM_SHARED((NPAD, 16), jnp.float32),
        pltpu.SemaphoreType.DMA((NB,)),
        pltpu.SemaphoreType.DMA((NB,)),
        pltpu.SemaphoreType.DMA((NB,)),
    ],
)
def _sc_pass_a(edge_hbm, asrc_hbm, adst_hbm, maxs_hbm, maxd_hbm,
               zeros_hbm, ex_hbm, den0_hbm, den1_hbm,
               edge_all, arows, drows, exbuf, m1, m2, denom_sp,
               gsem, osem, esem):
    cid = lax.axis_index("c")
    sid = lax.axis_index("s")
    wid = cid * 16 + sid
    row0 = wid * NCHUNKS

    pltpu.sync_copy(zeros_hbm.at[pl.ds(sid * ROWS_PER_TILE, ROWS_PER_TILE)],
                    denom_sp.at[pl.ds(sid * ROWS_PER_TILE, ROWS_PER_TILE)])
    pltpu.sync_copy(maxs_hbm.at[0], m1)
    pltpu.sync_copy(maxd_hbm.at[0], m2)
    pltpu.sync_copy(edge_hbm.at[pl.ds(row0, NCHUNKS)], edge_all)
    sub = _lrelu(m1[...] + m2[...])
    plsc.subcore_barrier()

    def superiter(t, carry):
        gc = []
        for b in range(NB):
            c = t * NB + b
            gc.append((
                pltpu.async_copy(asrc_hbm.at[edge_all.at[c, 0]], arows.at[b],
                                 gsem.at[b]),
                pltpu.async_copy(adst_hbm.at[edge_all.at[c, 1]], drows.at[b],
                                 gsem.at[b]),
            ))
        oc = []
        for b in range(NB):
            c = t * NB + b
            gc[b][0].wait()
            gc[b][1].wait()

            @plsc.parallel_loop(0, KC, unroll=8)
            def _edges(e, _b=b):
                v = _lrelu(arows[_b, e, :] + drows[_b, e, :])
                exbuf[_b, e, :] = jnp.exp(v - sub)

            base = pl.multiple_of(wid * EPW + c * KC, 8)
            oc.append((
                pltpu.async_copy(exbuf.at[b], denom_sp.at[edge_all.at[c, 1]],
                                 osem.at[b], add=True),
                pltpu.async_copy(exbuf.at[b], ex_hbm.at[pl.ds(base, KC)],
                                 esem.at[b]),
            ))
        for b in range(NB):
            oc[b][0].wait()
            oc[b][1].wait()
        return carry

    lax.fori_loop(0, SUPERS, superiter, 0)
    plsc.subcore_barrier()

    @pl.when(cid == 0)
    def _():
        pltpu.sync_copy(
            denom_sp.at[pl.ds(sid * ROWS_PER_TILE, ROWS_PER_TILE)],
            den0_hbm.at[pl.ds(sid * ROWS_PER_TILE, ROWS_PER_TILE)])

    @pl.when(cid == 1)
    def _():
        pltpu.sync_copy(
            denom_sp.at[pl.ds(sid * ROWS_PER_TILE, ROWS_PER_TILE)],
            den1_hbm.at[pl.ds(sid * ROWS_PER_TILE, ROWS_PER_TILE)])


def _make_sc_pass_b(width):
    nheads = width // 16

    unroll = 8 if nheads == 1 else 2
    nb = 3 if width == 128 else 6
    supers = NCHUNKS // nb

    @functools.partial(
        pl.kernel,
        mesh=_MESH,
        compiler_params=_SC_PARAMS,
        out_type=[
            jax.ShapeDtypeStruct((2, NPAD, width), jnp.float32),
            jax.ShapeDtypeStruct((2, NPAD, 16), jnp.float32),
        ],
        scratch_types=[
            pltpu.VMEM((2, nb, 2, KC), jnp.int32),
            pltpu.VMEM((nb, KC, width), jnp.float32),
            pltpu.VMEM((nb, KC, 16), jnp.float32),
            pltpu.VMEM((nb, KC, 16), jnp.float32),
            pltpu.VMEM_SHARED((NPAD, width), jnp.float32),
            pltpu.SemaphoreType.DMA((nb,)),
            pltpu.SemaphoreType.DMA((nb,)),
            pltpu.SemaphoreType.DMA((2,)),
        ],
    )
    def _sc_pass_b(edge_hbm, ex_hbm, den0_hbm, den1_hbm, h_hbm, zeros_hbm,
                   out_hbm, invc_hbm, eidx, hrows, exrows, invrows, out_sp,
                   gsem, osem, isem):
        cid = lax.axis_index("c")
        sid = lax.axis_index("s")
        wid = cid * 16 + sid
        row0 = wid * NCHUNKS

        pltpu.sync_copy(zeros_hbm.at[pl.ds(sid * ROWS_PER_TILE, ROWS_PER_TILE)],
                        out_sp.at[pl.ds(sid * ROWS_PER_TILE, ROWS_PER_TILE)])
        for s in range(ROWS_PER_TILE // 80):
            rs = sid * ROWS_PER_TILE + s * 80
            pltpu.sync_copy(den0_hbm.at[pl.ds(rs, 80)],
                            exrows.at[0, pl.ds(0, 80)])
            pltpu.sync_copy(den1_hbm.at[pl.ds(rs, 80)],
                            invrows.at[0, pl.ds(0, 80)])

            @plsc.parallel_loop(0, 80, unroll=4)
            def _rows(r, _s=s):
                invrows[0, r, :] = 1.0 / (
                    exrows[0, r, :] + invrows[0, r, :] + 1e-16)

            pltpu.sync_copy(invrows.at[0, pl.ds(0, 80)],
                            invc_hbm.at[cid, pl.ds(rs, 80)])
        pltpu.sync_copy(edge_hbm.at[pl.ds(row0, nb)], eidx.at[0])
        plsc.subcore_barrier()

        def superiter(t, p):
            nxt = pltpu.async_copy(
                edge_hbm.at[pl.ds(row0 + (t + 1) * nb, nb)], eidx.at[1 - p],
                isem.at[1 - p])
            gc = []
            for b in range(nb):
                c = t * nb + b
                base = pl.multiple_of(wid * EPW + c * KC, 8)
                gc.append((
                    pltpu.async_copy(h_hbm.at[eidx.at[p, b, 0]], hrows.at[b],
                                     gsem.at[b]),
                    pltpu.async_copy(invc_hbm.at[cid].at[eidx.at[p, b, 1]],
                                     invrows.at[b], gsem.at[b]),
                    pltpu.async_copy(ex_hbm.at[pl.ds(base, KC)], exrows.at[b],
                                     gsem.at[b]),
                ))
            oc = []
            for b in range(nb):
                for g in gc[b]:
                    g.wait()

                @plsc.parallel_loop(0, KC, unroll=unroll)
                def _edges(e, _b=b):
                    coef = exrows[_b, e, :] * invrows[_b, e, :]
                    for hd in range(nheads):
                        hv = hrows[_b, e, pl.ds(hd * 16, 16)]
                        hrows[_b, e, pl.ds(hd * 16, 16)] = hv * _splat(coef, hd)

                oc.append(
                    pltpu.async_copy(hrows.at[b],
                                     out_sp.at[eidx.at[p, b, 1]],
                                     osem.at[b], add=True))
            for b in range(nb):
                oc[b].wait()
            nxt.wait()

        def pair(u, carry):
            superiter(2 * u, 0)
            superiter(2 * u + 1, 1)
            return carry

        lax.fori_loop(0, supers // 2, pair, 0)
        plsc.subcore_barrier()
        pltpu.sync_copy(out_sp.at[pl.ds(sid * ROWS_PER_TILE, ROWS_PER_TILE)],
                        out_hbm.at[cid, pl.ds(sid * ROWS_PER_TILE, ROWS_PER_TILE)])

    return _sc_pass_b


_sc_pass_b128 = _make_sc_pass_b(128)
_sc_pass_b16 = _make_sc_pass_b(16)


def kernel(x, edge_index, W1, a_src1, a_dst1, b1, W2, a_src2, a_dst2, b2):
    n = N_NODES
    loop = jnp.arange(n, dtype=jnp.int32)
    pad_e = EP - (E_RAW + n)
    padv = n + (jnp.arange(pad_e, dtype=jnp.int32) % (NPAD - n))
    srcp = jnp.concatenate([edge_index[0], loop, padv]).reshape(EP // KC, KC)
    dstp = jnp.concatenate([edge_index[1], loop, padv]).reshape(EP // KC, KC)
    edge2d = jnp.concatenate(
        [jnp.stack([srcp, dstp], axis=1),
         jnp.zeros((6, 2, KC), jnp.int32)], axis=0)

    xp = jnp.zeros((NPAD, 128), jnp.float32).at[:n].set(x)

    hd_ids = jnp.repeat(jnp.arange(8), 16)
    sel = (hd_ids[:, None] == jnp.arange(8)[None, :])
    ms1 = jnp.pad(jnp.where(sel, a_src1.reshape(-1)[:, None], 0.0), ((0, 0), (0, 8)))
    md1 = jnp.pad(jnp.where(sel, a_dst1.reshape(-1)[:, None], 0.0), ((0, 0), (0, 8)))

    w2p = jnp.pad(W2, ((0, 0), (0, 16 - W2.shape[1])))
    ms2 = jnp.pad(W2 @ a_src2.T, ((0, 0), (0, 15)))
    md2 = jnp.pad(W2 @ a_dst2.T, ((0, 0), (0, 15)))
    b1b = jnp.broadcast_to(b1[None, :], (8, 128))
    b2b = jnp.broadcast_to(jnp.pad(b2, (0, 16 - b2.shape[0]))[None, :], (8, 16))
    zeros128 = jnp.zeros((NPAD, 128), jnp.float32)
    zeros16 = jnp.zeros((NPAD, 16), jnp.float32)

    h1, as1, ad1, maxs1, maxd1 = _tc1(xp, W1, ms1, md1)
    ex1, dn0_1, dn1_1 = _sc_pass_a(edge2d, as1, ad1, maxs1, maxd1, zeros16)
    out1, _ = _sc_pass_b128(edge2d, ex1, dn0_1, dn1_1, h1, zeros128)

    h2, as2, ad2, maxs2, maxd2 = _tc3(out1, b1b, w2p, ms2, md2)
    ex2, dn0_2, dn1_2 = _sc_pass_a(edge2d, as2, ad2, maxs2, maxd2, zeros16)
    out2, _ = _sc_pass_b16(edge2d, ex2, dn0_2, dn1_2, h2, zeros16)

    res = _tc4(out2, b2b)
    return res[:n, :7]
